# baseline (speedup 1.0000x reference)
import functools
import jax
import jax.numpy as jnp
from jax import lax
from jax.experimental import pallas as pl
from jax.experimental.pallas import tpu as pltpu
from jax.experimental.pallas import tpu_sc as plsc

N = 10000
E = 320000
DIN = 128
EDIM = 16
DOUT = 128
NCLS = 11

NC = 2
NS = 16
NW = NC * NS
CH = 64
CHUNKS = 160
CH3 = 128
CHUNKS3 = 80
EPT = CH * CHUNKS
EP = EPT * NW
NP = 10240
RPT = NP // NS
KB = CHUNKS // 2 - 1
KB3 = CHUNKS3 // 2 - 1
CT = EP // (NS * CH)
CT3 = EP // (NS * CH3)
EP8 = EP // 8

_MESH = plsc.VectorSubcoreMesh(core_axis_name="c", subcore_axis_name="s",
                               num_cores=1)
_F32 = jnp.float32
_HIGH = lax.Precision.HIGHEST


def _dot(a, b):
    return jnp.dot(a, b, preferred_element_type=_F32, precision=_HIGH)


def _edge_msg_body(e_ref, w_ref, b_ref, o_ref):
    o_ref[...] = _dot(e_ref[...], w_ref[...]) + b_ref[...]


def _node_msg_body(h_ref, w_ref, o_ref):
    nb = h_ref.shape[0]
    hw = _dot(h_ref[...], w_ref[...])
    col = lax.broadcasted_iota(jnp.int32, (nb, DIN), 1)
    wide = jnp.concatenate(
        [hw, jnp.zeros((nb, DIN - EDIM), _F32)], axis=1)
    o_ref[...] = jnp.where(col == EDIM, 1.0, wide)


def _layer1_apply_body(h0_ref, s1_ref, wa1t_ref, wa1b_ref, ba1_ref,
                       wp2_ref, wq2_ref, bm2_ref, h1_ref, p_ref, qb_ref):
    acc = s1_ref[...]
    cnt = acc[:, EDIM]
    inv = 1.0 / jnp.maximum(cnt, 1.0)
    hn1 = acc[:, :EDIM] * inv[:, None]
    h1 = jax.nn.relu(_dot(h0_ref[...], wa1t_ref[...]) +
                     _dot(hn1, wa1b_ref[...]) + ba1_ref[...])
    h1_ref[...] = h1
    p_ref[...] = _dot(h1, wp2_ref[...])
    qb_ref[...] = _dot(h1, wq2_ref[...]) + bm2_ref[...]


def _layer2_apply_body(h1_ref, s1_ref, s2_ref, qb_ref, wa2t_ref, wa2b_ref,
                       ba2_ref, wpt_ref, wpb_ref, bp_ref, ab_ref):
    nb = h1_ref.shape[0]
    cnt = s1_ref[:, EDIM]
    inv = 1.0 / jnp.maximum(cnt, 1.0)
    gate = jnp.minimum(cnt, 1.0)
    s2t = s2_ref[...]
    hn2 = s2t * inv[:, None] + qb_ref[...] * gate[:, None]
    h2 = jax.nn.relu(_dot(h1_ref[...], wa2t_ref[...]) +
                     _dot(hn2, wa2b_ref[...]) + ba2_ref[...])
    a = _dot(h2, wpt_ref[...]) + bp_ref[...]
    b = _dot(h2, wpb_ref[...])
    ab_ref[...] = jnp.concatenate(
        [a, b, jnp.zeros((nb, DIN - 2 * EDIM), _F32)], axis=1)


def _zero_rows(buf, rows, width):
    zero = jnp.zeros((16,), _F32)
    for r in range(rows):
        for c in range(width // 16):
            buf[r, pl.ds(c * 16, 16)] = zero


@functools.partial(
    pl.kernel,
    out_type=jax.ShapeDtypeStruct((NP, DIN), _F32),
    mesh=_MESH,
    scratch_types=[
        pltpu.VMEM((CH,), jnp.int32),
        pltpu.VMEM((CH,), jnp.int32),
        pltpu.VMEM((CH,), jnp.int32),
        pltpu.VMEM((CH,), jnp.int32),
        pltpu.VMEM((CH, DIN), _F32),
        pltpu.VMEM((CH, DIN), _F32),
        pltpu.VMEM((CH // 8, 8 * EDIM), _F32),
        pltpu.VMEM((CH // 8, 8 * EDIM), _F32),
        pltpu.VMEM_SHARED((NP, DIN), _F32),
        pltpu.SemaphoreType.DMA,
        pltpu.SemaphoreType.DMA,
        pltpu.SemaphoreType.DMA,
        pltpu.SemaphoreType.DMA,
        pltpu.SemaphoreType.DMA,
        pltpu.SemaphoreType.DMA,
        pltpu.SemaphoreType.DMA,
        pltpu.SemaphoreType.DMA,
        pltpu.SemaphoreType.DMA,
        pltpu.SemaphoreType.DMA,
    ],
)
def _sc_layer1(hc_hbm, ew_hbm, src_hbm, dst_hbm, s1_hbm,
               sidx0, sidx1, didx0, didx1, pbuf0, pbuf1, ebuf0, ebuf1,
               acc_sp, g0, g1, s0, s1, e0, e1, si0, si1, di0, di1):
    cid = lax.axis_index("c")
    sid = lax.axis_index("s")
    wid = cid * NS + sid
    _zero_rows(pbuf0, CH, DIN)
    _zero_rows(pbuf1, CH, DIN)
    row0 = sid * RPT
    for r in range(RPT // CH):
        pltpu.sync_copy(pbuf0, acc_sp.at[pl.ds(row0 + r * CH, CH)])
    koff = sid * CT
    cl = CT
    kbl = cl // 2 - 1
    ebase = koff * CH
    ebase8 = koff * 8
    pltpu.sync_copy(src_hbm.at[pl.ds(ebase, CH)], sidx0)
    plsc.subcore_barrier()

    def fi_src(k, buf, sem):
        pltpu.async_copy(src_hbm.at[pl.ds(ebase + k * CH, CH)], buf, sem)

    def fi_dst(k, buf, sem):
        pltpu.async_copy(dst_hbm.at[pl.ds(ebase + k * CH, CH)], buf, sem)

    def w_idx(buf, sem):
        pltpu.make_async_copy(src_hbm.at[pl.ds(0, CH)], buf, sem).wait()

    def gather(sbuf, buf, sem):
        pltpu.async_copy(hc_hbm.at[sbuf], buf, sem)

    def scatter(dbuf, buf, sem):
        pltpu.async_copy(buf, acc_sp.at[dbuf], sem, add=True)

    def eload(k, buf, sem):
        pltpu.async_copy(ew_hbm.at[pl.ds(ebase8 + k * (CH // 8), CH // 8)],
                         buf, sem)

    def wait_p(buf, sem):
        pltpu.make_async_copy(hc_hbm.at[pl.ds(0, CH)], buf, sem).wait()

    def wait_e(buf, sem):
        pltpu.make_async_copy(ew_hbm.at[pl.ds(0, CH // 8)], buf, sem).wait()

    def addin(pbuf, ebuf):
        for r in range(CH):
            pbuf[r, pl.ds(0, EDIM)] = (
                pbuf[r, pl.ds(0, EDIM)]
                + ebuf[r // 8, pl.ds((r % 8) * EDIM, EDIM)])

    gather(sidx0, pbuf0, g0)
    fi_dst(0, didx0, di0)
    fi_src(1, sidx1, si1)
    eload(0, ebuf0, e0)
    eload(1, ebuf1, e1)
    pltpu.async_copy(pbuf1, acc_sp.at[sidx0], s1, add=True)

    def body(kk, carry):
        k0 = 2 * kk
        k1 = k0 + 1
        wait_p(pbuf0, g0)
        fi_src(k0 + 2, sidx0, si0)
        wait_e(ebuf0, e0)
        addin(pbuf0, ebuf0)
        eload(k0 + 2, ebuf0, e0)
        w_idx(didx0, di0)
        scatter(didx0, pbuf0, s0)
        wait_p(pbuf1, s1)
        fi_dst(k1, didx1, di1)
        w_idx(sidx1, si1)
        gather(sidx1, pbuf1, g1)
        wait_p(pbuf1, g1)
        fi_src(k1 + 2, sidx1, si1)
        wait_e(ebuf1, e1)
        addin(pbuf1, ebuf1)
        eload(k1 + 2, ebuf1, e1)
        w_idx(didx1, di1)
        scatter(didx1, pbuf1, s1)
        wait_p(pbuf0, s0)
        fi_dst(k0 + 2, didx0, di0)
        w_idx(sidx0, si0)
        gather(sidx0, pbuf0, g0)
        return carry

    lax.fori_loop(0, kbl, body, 0)
    wait_p(pbuf0, g0)
    wait_e(ebuf0, e0)
    addin(pbuf0, ebuf0)
    w_idx(didx0, di0)
    scatter(didx0, pbuf0, s0)
    wait_p(pbuf1, s1)
    fi_dst(cl - 1, didx1, di1)
    w_idx(sidx1, si1)
    gather(sidx1, pbuf1, g1)
    wait_p(pbuf1, g1)
    wait_e(ebuf1, e1)
    addin(pbuf1, ebuf1)
    w_idx(didx1, di1)
    scatter(didx1, pbuf1, s1)
    wait_p(pbuf0, s0)
    wait_p(pbuf1, s1)
    plsc.subcore_barrier()
    for r in range(RPT // CH):
        pltpu.sync_copy(acc_sp.at[pl.ds(row0 + r * CH, CH)], pbuf0)
        pltpu.sync_copy(pbuf0, s1_hbm.at[pl.ds(row0 + r * CH, CH)])


@functools.partial(
    pl.kernel,
    out_type=jax.ShapeDtypeStruct((NP, DOUT), _F32),
    mesh=_MESH,
    scratch_types=[
        pltpu.VMEM((CH,), jnp.int32),
        pltpu.VMEM((CH,), jnp.int32),
        pltpu.VMEM((CH,), jnp.int32),
        pltpu.VMEM((CH,), jnp.int32),
        pltpu.VMEM((CH, DOUT), _F32),
        pltpu.VMEM((CH, DOUT), _F32),
        pltpu.VMEM_SHARED((NP, DOUT), _F32),
        pltpu.SemaphoreType.DMA,
        pltpu.SemaphoreType.DMA,
        pltpu.SemaphoreType.DMA,
        pltpu.SemaphoreType.DMA,
        pltpu.SemaphoreType.DMA,
        pltpu.SemaphoreType.DMA,
        pltpu.SemaphoreType.DMA,
        pltpu.SemaphoreType.DMA,
    ],
)
def _sc_layer2(p_hbm, src_hbm, dst_hbm, s2_hbm,
               sidx0, sidx1, didx0, didx1, pbuf0, pbuf1,
               acc_sp, g0, g1, s0, s1, si0, si1, di0, di1):
    cid = lax.axis_index("c")
    sid = lax.axis_index("s")
    wid = cid * NS + sid
    _zero_rows(pbuf0, CH, DOUT)
    _zero_rows(pbuf1, CH, DOUT)
    row0 = sid * RPT
    for r in range(RPT // CH):
        pltpu.sync_copy(pbuf0, acc_sp.at[pl.ds(row0 + r * CH, CH)])
    koff = sid * CT
    cl = CT
    kbl = cl // 2 - 1
    ebase = koff * CH
    pltpu.sync_copy(src_hbm.at[pl.ds(ebase, CH)], sidx0)
    plsc.subcore_barrier()

    def fi_src(k, buf, sem):
        pltpu.async_copy(src_hbm.at[pl.ds(ebase + k * CH, CH)], buf, sem)

    def fi_dst(k, buf, sem):
        pltpu.async_copy(dst_hbm.at[pl.ds(ebase + k * CH, CH)], buf, sem)

    def w_idx(buf, sem):
        pltpu.make_async_copy(src_hbm.at[pl.ds(0, CH)], buf, sem).wait()

    def gather(sbuf, buf, sem):
        pltpu.async_copy(p_hbm.at[sbuf], buf, sem)

    def scatter(dbuf, buf, sem):
        pltpu.async_copy(buf, acc_sp.at[dbuf], sem, add=True)

    def wait_p(buf, sem):
        pltpu.make_async_copy(p_hbm.at[pl.ds(0, CH)], buf, sem).wait()

    gather(sidx0, pbuf0, g0)
    fi_dst(0, didx0, di0)
    fi_src(1, sidx1, si1)
    pltpu.async_copy(pbuf1, acc_sp.at[sidx0], s1, add=True)

    def body(kk, carry):
        k0 = 2 * kk
        k1 = k0 + 1
        wait_p(pbuf0, g0)
        fi_src(k0 + 2, sidx0, si0)
        w_idx(didx0, di0)
        scatter(didx0, pbuf0, s0)
        wait_p(pbuf1, s1)
        fi_dst(k1, didx1, di1)
        w_idx(sidx1, si1)
        gather(sidx1, pbuf1, g1)
        wait_p(pbuf1, g1)
        fi_src(k1 + 2, sidx1, si1)
        w_idx(didx1, di1)
        scatter(didx1, pbuf1, s1)
        wait_p(pbuf0, s0)
        fi_dst(k0 + 2, didx0, di0)
        w_idx(sidx0, si0)
        gather(sidx0, pbuf0, g0)
        return carry

    lax.fori_loop(0, kbl, body, 0)
    wait_p(pbuf0, g0)
    w_idx(didx0, di0)
    scatter(didx0, pbuf0, s0)
    wait_p(pbuf1, s1)
    fi_dst(cl - 1, didx1, di1)
    w_idx(sidx1, si1)
    gather(sidx1, pbuf1, g1)
    wait_p(pbuf1, g1)
    w_idx(didx1, di1)
    scatter(didx1, pbuf1, s1)
    wait_p(pbuf0, s0)
    wait_p(pbuf1, s1)
    plsc.subcore_barrier()
    for r in range(RPT // CH):
        pltpu.sync_copy(acc_sp.at[pl.ds(row0 + r * CH, CH)], pbuf0)
        pltpu.sync_copy(pbuf0, s2_hbm.at[pl.ds(row0 + r * CH, CH)])


@functools.partial(
    pl.kernel,
    out_type=jax.ShapeDtypeStruct((EP8, 8 * EDIM), _F32),
    mesh=_MESH,
    scratch_types=[
        pltpu.VMEM((CT3, CH3), jnp.int32),
        pltpu.VMEM((CT3, CH3), jnp.int32),
        pltpu.VMEM((CH3, DIN), _F32),
        pltpu.VMEM((CH3, DIN), _F32),
        pltpu.VMEM((CH3, DIN), _F32),
        pltpu.VMEM((CH3, DIN), _F32),
        pltpu.VMEM((CH3 // 8, 8 * EDIM), _F32),
        pltpu.VMEM((CH3 // 8, 8 * EDIM), _F32),
        pltpu.SemaphoreType.DMA,
        pltpu.SemaphoreType.DMA,
        pltpu.SemaphoreType.DMA,
        pltpu.SemaphoreType.DMA,
        pltpu.SemaphoreType.DMA,
        pltpu.SemaphoreType.DMA,
    ],
)
def _sc_score(ab_hbm, src_hbm, dst_hbm, out_hbm,
              sidx, didx, u0, u1, v0, v1, o0, o1,
              gu0, gu1, gv0, gv1, so0, so1):
    cid = lax.axis_index("c")
    sid = lax.axis_index("s")
    wid = cid * NS + sid
    koff = sid * CT3
    cl = CT3
    kbl = cl // 2 - 1
    pltpu.sync_copy(src_hbm.at[pl.ds(koff, CT3)], sidx)
    pltpu.sync_copy(dst_hbm.at[pl.ds(koff, CT3)], didx)
    ebase8 = koff * (CH3 // 8)

    def gu(k, buf, sem):
        pltpu.async_copy(ab_hbm.at[sidx.at[k]], buf, sem)

    def gv(k, buf, sem):
        pltpu.async_copy(ab_hbm.at[didx.at[k]], buf, sem)

    def ostore(k, buf, sem):
        pltpu.async_copy(
            buf, out_hbm.at[pl.ds(ebase8 + k * (CH3 // 8), CH3 // 8)], sem)

    def wait_w(buf, sem):
        pltpu.make_async_copy(ab_hbm.at[pl.ds(0, CH3)], buf, sem).wait()

    def wait_o(buf, sem):
        pltpu.make_async_copy(out_hbm.at[pl.ds(0, CH3 // 8)], buf, sem).wait()

    def combine(ob, ub, vb):
        for r in range(CH3):
            ob[r // 8, pl.ds((r % 8) * EDIM, EDIM)] = (
                ub[r, pl.ds(0, EDIM)] + vb[r, pl.ds(EDIM, EDIM)])

    gu(0, u0, gu0)
    gv(0, v0, gv0)
    gu(1, u1, gu1)
    gv(1, v1, gv1)
    wait_w(u0, gu0)
    wait_w(v0, gv0)
    combine(o0, u0, v0)
    ostore(0, o0, so0)
    gu(2, u0, gu0)
    gv(2, v0, gv0)
    wait_w(u1, gu1)
    wait_w(v1, gv1)
    combine(o1, u1, v1)
    ostore(1, o1, so1)
    gu(3, u1, gu1)
    gv(3, v1, gv1)

    def body(kk, carry):
        k0 = 2 * kk
        k1 = k0 + 1
        wait_w(u0, gu0)
        wait_w(v0, gv0)
        wait_o(o0, so0)
        combine(o0, u0, v0)
        ostore(k0, o0, so0)
        gu(k0 + 2, u0, gu0)
        gv(k0 + 2, v0, gv0)
        wait_w(u1, gu1)
        wait_w(v1, gv1)
        wait_o(o1, so1)
        combine(o1, u1, v1)
        ostore(k1, o1, so1)
        gu(k1 + 2, u1, gu1)
        gv(k1 + 2, v1, gv1)
        return carry

    lax.fori_loop(1, kbl, body, 0)
    wait_w(u0, gu0)
    wait_w(v0, gv0)
    wait_o(o0, so0)
    combine(o0, u0, v0)
    ostore(cl - 2, o0, so0)
    wait_w(u1, gu1)
    wait_w(v1, gv1)
    wait_o(o1, so1)
    combine(o1, u1, v1)
    ostore(cl - 1, o1, so1)
    wait_o(o0, so0)
    wait_o(o1, so1)


def kernel(nfeats, efeats, edge_index, Wm1, bm1, Wa1, ba1,
           Wm2, bm2, Wa2, ba2, Wp, bp):
    h0 = nfeats.reshape(N, DIN)
    h0p = jnp.pad(h0, ((0, NP - N), (0, 0)))
    srcp = jnp.concatenate(
        [edge_index[0], jnp.full((EP - E,), N, jnp.int32)])
    dstp = jnp.concatenate(
        [edge_index[1], jnp.full((EP - E,), N, jnp.int32)])
    src2d = srcp.reshape(-1, CH3)
    dst2d = dstp.reshape(-1, CH3)

    wm1t, wm1b = Wm1[:DIN], Wm1[DIN:]
    wa1t, wa1b = Wa1[:DIN], Wa1[DIN:]
    wp2 = Wm2[:EDIM] + 0.5 * Wm2[EDIM:]
    wq2 = 0.5 * Wm2[EDIM:]
    wa2t, wa2b = Wa2[:EDIM], Wa2[EDIM:]
    wpt = jnp.pad(Wp[:DOUT], ((0, 0), (0, 16 - NCLS)))
    wpb = jnp.pad(Wp[DOUT:], ((0, 0), (0, 16 - NCLS)))
    bpp = jnp.pad(bp, (0, 16 - NCLS)).reshape(1, 16)
    bm1r = bm1.reshape(1, EDIM)
    ba1r = ba1.reshape(1, EDIM)
    bm2r = bm2.reshape(1, DOUT)
    ba2r = ba2.reshape(1, DOUT)

    e8 = jnp.pad(efeats.reshape(E // 8, 8 * EDIM),
                 ((0, EP8 - E // 8), (0, 0)))
    wblk = jnp.kron(jnp.eye(8, dtype=_F32), wm1b)
    btile = jnp.tile(bm1, 8).reshape(1, 8 * EDIM)
    be8 = 320
    ew = pl.pallas_call(
        _edge_msg_body,
        grid=(EP8 // be8,),
        in_specs=[pl.BlockSpec((be8, 8 * EDIM), lambda i: (i, 0)),
                  pl.BlockSpec((8 * EDIM, 8 * EDIM), lambda i: (0, 0)),
                  pl.BlockSpec((1, 8 * EDIM), lambda i: (0, 0))],
        out_specs=pl.BlockSpec((be8, 8 * EDIM), lambda i: (i, 0)),
        out_shape=jax.ShapeDtypeStruct((EP8, 8 * EDIM), _F32),
    )(e8, wblk, btile)

    nb = 1280
    hc = pl.pallas_call(
        _node_msg_body,
        grid=(NP // nb,),
        in_specs=[pl.BlockSpec((nb, DIN), lambda i: (i, 0)),
                  pl.BlockSpec((DIN, EDIM), lambda i: (0, 0))],
        out_specs=pl.BlockSpec((nb, DIN), lambda i: (i, 0)),
        out_shape=jax.ShapeDtypeStruct((NP, DIN), _F32),
    )(h0p, wm1t)

    s1 = _sc_layer1(hc, ew, srcp, dstp)

    h1, pmat, qb = pl.pallas_call(
        _layer1_apply_body,
        grid=(NP // nb,),
        in_specs=[pl.BlockSpec((nb, DIN), lambda i: (i, 0)),
                  pl.BlockSpec((nb, DIN), lambda i: (i, 0)),
                  pl.BlockSpec((DIN, EDIM), lambda i: (0, 0)),
                  pl.BlockSpec((EDIM, EDIM), lambda i: (0, 0)),
                  pl.BlockSpec((1, EDIM), lambda i: (0, 0)),
                  pl.BlockSpec((EDIM, DOUT), lambda i: (0, 0)),
                  pl.BlockSpec((EDIM, DOUT), lambda i: (0, 0)),
                  pl.BlockSpec((1, DOUT), lambda i: (0, 0))],
        out_specs=[pl.BlockSpec((nb, EDIM), lambda i: (i, 0)),
                   pl.BlockSpec((nb, DOUT), lambda i: (i, 0)),
                   pl.BlockSpec((nb, DOUT), lambda i: (i, 0))],
        out_shape=[jax.ShapeDtypeStruct((NP, EDIM), _F32),
                   jax.ShapeDtypeStruct((NP, DOUT), _F32),
                   jax.ShapeDtypeStruct((NP, DOUT), _F32)],
    )(h0p, s1, wa1t, wa1b, ba1r, wp2, wq2, bm2r)

    s2 = _sc_layer2(pmat, srcp, dstp)

    ab = pl.pallas_call(
        _layer2_apply_body,
        grid=(NP // nb,),
        in_specs=[pl.BlockSpec((nb, EDIM), lambda i: (i, 0)),
                  pl.BlockSpec((nb, DIN), lambda i: (i, 0)),
                  pl.BlockSpec((nb, DOUT), lambda i: (i, 0)),
                  pl.BlockSpec((nb, DOUT), lambda i: (i, 0)),
                  pl.BlockSpec((EDIM, DOUT), lambda i: (0, 0)),
                  pl.BlockSpec((DOUT, DOUT), lambda i: (0, 0)),
                  pl.BlockSpec((1, DOUT), lambda i: (0, 0)),
                  pl.BlockSpec((DOUT, 16), lambda i: (0, 0)),
                  pl.BlockSpec((DOUT, 16), lambda i: (0, 0)),
                  pl.BlockSpec((1, 16), lambda i: (0, 0))],
        out_specs=pl.BlockSpec((nb, DIN), lambda i: (i, 0)),
        out_shape=jax.ShapeDtypeStruct((NP, DIN), _F32),
    )(h1, s1, s2, qb, wa2t, wa2b, ba2r, wpt, wpb, bpp)

    score = _sc_score(ab, src2d, dst2d).reshape(EP, EDIM)
    return score[:E, :NCLS]


# speedup vs baseline: 2.9132x; 2.9132x over previous
import functools
import jax
import jax.numpy as jnp
from jax import lax
from jax.experimental import pallas as pl
from jax.experimental.pallas import tpu as pltpu
from jax.experimental.pallas import tpu_sc as plsc

N = 10000
E = 320000
DIN = 128
EDIM = 16
DOUT = 128
NCLS = 11

NC = 2
NS = 16
NW = NC * NS
CH = 64
CHUNKS = 160
CH3 = 128
CHUNKS3 = 80
EPT = CH * CHUNKS
EP = EPT * NW
NP = 10240
RPT = NP // NS
KB = CHUNKS // 2 - 1
KB3 = CHUNKS3 // 2 - 1
CA = 160
CB = 160
CA3 = 80
CB3 = 80
EP8 = EP // 8

_MESH = plsc.VectorSubcoreMesh(core_axis_name="c", subcore_axis_name="s")
_F32 = jnp.float32
_HIGH = lax.Precision.HIGHEST


def _dot(a, b):
    return jnp.dot(a, b, preferred_element_type=_F32, precision=_HIGH)


def _edge_msg_body(e_ref, w_ref, b_ref, o_ref):
    o_ref[...] = _dot(e_ref[...], w_ref[...]) + b_ref[...]


def _node_msg_body(h_ref, w_ref, o_ref):
    nb = h_ref.shape[0]
    hw = _dot(h_ref[...], w_ref[...])
    col = lax.broadcasted_iota(jnp.int32, (nb, DIN), 1)
    wide = jnp.concatenate(
        [hw, jnp.zeros((nb, DIN - EDIM), _F32)], axis=1)
    o_ref[...] = jnp.where(col == EDIM, 1.0, wide)


def _layer1_apply_body(h0_ref, s1_ref, wa1t_ref, wa1b_ref, ba1_ref,
                       wp2_ref, wq2_ref, bm2_ref, h1_ref, p_ref, qb_ref):
    acc = s1_ref[0] + s1_ref[1]
    cnt = acc[:, EDIM]
    inv = 1.0 / jnp.maximum(cnt, 1.0)
    hn1 = acc[:, :EDIM] * inv[:, None]
    h1 = jax.nn.relu(_dot(h0_ref[...], wa1t_ref[...]) +
                     _dot(hn1, wa1b_ref[...]) + ba1_ref[...])
    h1_ref[...] = h1
    p_ref[...] = _dot(h1, wp2_ref[...])
    qb_ref[...] = _dot(h1, wq2_ref[...]) + bm2_ref[...]


def _layer2_apply_body(h1_ref, s1_ref, s2_ref, qb_ref, wa2t_ref, wa2b_ref,
                       ba2_ref, wpt_ref, wpb_ref, bp_ref, ab_ref):
    nb = h1_ref.shape[0]
    cnt = s1_ref[0][:, EDIM] + s1_ref[1][:, EDIM]
    inv = 1.0 / jnp.maximum(cnt, 1.0)
    gate = jnp.minimum(cnt, 1.0)
    s2t = s2_ref[0] + s2_ref[1]
    hn2 = s2t * inv[:, None] + qb_ref[...] * gate[:, None]
    h2 = jax.nn.relu(_dot(h1_ref[...], wa2t_ref[...]) +
                     _dot(hn2, wa2b_ref[...]) + ba2_ref[...])
    a = _dot(h2, wpt_ref[...]) + bp_ref[...]
    b = _dot(h2, wpb_ref[...])
    ab_ref[...] = jnp.concatenate(
        [a, b, jnp.zeros((nb, DIN - 2 * EDIM), _F32)], axis=1)


def _zero_rows(buf, rows, width):
    zero = jnp.zeros((16,), _F32)
    for r in range(rows):
        for c in range(width // 16):
            buf[r, pl.ds(c * 16, 16)] = zero


@functools.partial(
    pl.kernel,
    out_type=jax.ShapeDtypeStruct((NC * NP, DIN), _F32),
    mesh=_MESH,
    scratch_types=[
        pltpu.VMEM((CH,), jnp.int32),
        pltpu.VMEM((CH,), jnp.int32),
        pltpu.VMEM((CH,), jnp.int32),
        pltpu.VMEM((CH,), jnp.int32),
        pltpu.VMEM((CH, DIN), _F32),
        pltpu.VMEM((CH, DIN), _F32),
        pltpu.VMEM((CH // 8, 8 * EDIM), _F32),
        pltpu.VMEM((CH // 8, 8 * EDIM), _F32),
        pltpu.VMEM_SHARED((NP, DIN), _F32),
        pltpu.SemaphoreType.DMA,
        pltpu.SemaphoreType.DMA,
        pltpu.SemaphoreType.DMA,
        pltpu.SemaphoreType.DMA,
        pltpu.SemaphoreType.DMA,
        pltpu.SemaphoreType.DMA,
        pltpu.SemaphoreType.DMA,
        pltpu.SemaphoreType.DMA,
        pltpu.SemaphoreType.DMA,
        pltpu.SemaphoreType.DMA,
    ],
)
def _sc_layer1(hc_hbm, ew_hbm, src_hbm, dst_hbm, s1_hbm,
               sidx0, sidx1, didx0, didx1, pbuf0, pbuf1, ebuf0, ebuf1,
               acc_sp, g0, g1, s0, s1, e0, e1, si0, si1, di0, di1):
    cid = lax.axis_index("c")
    sid = lax.axis_index("s")
    wid = cid * NS + sid
    _zero_rows(pbuf0, CH, DIN)
    _zero_rows(pbuf1, CH, DIN)
    row0 = sid * RPT
    for r in range(RPT // CH):
        pltpu.sync_copy(pbuf0, acc_sp.at[pl.ds(row0 + r * CH, CH)])
    koff = jnp.where(cid == 0, sid * CA, NS * CA + sid * CB)
    cl = jnp.where(cid == 0, CA, CB)
    kbl = cl // 2 - 1
    ebase = koff * CH
    ebase8 = koff * 8
    pltpu.sync_copy(src_hbm.at[pl.ds(ebase, CH)], sidx0)
    plsc.subcore_barrier()

    def fi_src(k, buf, sem):
        pltpu.async_copy(src_hbm.at[pl.ds(ebase + k * CH, CH)], buf, sem)

    def fi_dst(k, buf, sem):
        pltpu.async_copy(dst_hbm.at[pl.ds(ebase + k * CH, CH)], buf, sem)

    def w_idx(buf, sem):
        pltpu.make_async_copy(src_hbm.at[pl.ds(0, CH)], buf, sem).wait()

    def gather(sbuf, buf, sem):
        pltpu.async_copy(hc_hbm.at[sbuf], buf, sem)

    def scatter(dbuf, buf, sem):
        pltpu.async_copy(buf, acc_sp.at[dbuf], sem, add=True)

    def eload(k, buf, sem):
        pltpu.async_copy(ew_hbm.at[pl.ds(ebase8 + k * (CH // 8), CH // 8)],
                         buf, sem)

    def wait_p(buf, sem):
        pltpu.make_async_copy(hc_hbm.at[pl.ds(0, CH)], buf, sem).wait()

    def wait_e(buf, sem):
        pltpu.make_async_copy(ew_hbm.at[pl.ds(0, CH // 8)], buf, sem).wait()

    def addin(pbuf, ebuf):
        for r in range(CH):
            pbuf[r, pl.ds(0, EDIM)] = (
                pbuf[r, pl.ds(0, EDIM)]
                + ebuf[r // 8, pl.ds((r % 8) * EDIM, EDIM)])

    gather(sidx0, pbuf0, g0)
    fi_dst(0, didx0, di0)
    fi_src(1, sidx1, si1)
    eload(0, ebuf0, e0)
    eload(1, ebuf1, e1)
    pltpu.async_copy(pbuf1, acc_sp.at[sidx0], s1, add=True)

    def body(kk, carry):
        k0 = 2 * kk
        k1 = k0 + 1
        wait_p(pbuf0, g0)
        fi_src(k0 + 2, sidx0, si0)
        wait_e(ebuf0, e0)
        addin(pbuf0, ebuf0)
        eload(k0 + 2, ebuf0, e0)
        w_idx(didx0, di0)
        scatter(didx0, pbuf0, s0)
        wait_p(pbuf1, s1)
        fi_dst(k1, didx1, di1)
        w_idx(sidx1, si1)
        gather(sidx1, pbuf1, g1)
        wait_p(pbuf1, g1)
        fi_src(k1 + 2, sidx1, si1)
        wait_e(ebuf1, e1)
        addin(pbuf1, ebuf1)
        eload(k1 + 2, ebuf1, e1)
        w_idx(didx1, di1)
        scatter(didx1, pbuf1, s1)
        wait_p(pbuf0, s0)
        fi_dst(k0 + 2, didx0, di0)
        w_idx(sidx0, si0)
        gather(sidx0, pbuf0, g0)
        return carry

    lax.fori_loop(0, kbl, body, 0)
    wait_p(pbuf0, g0)
    wait_e(ebuf0, e0)
    addin(pbuf0, ebuf0)
    w_idx(didx0, di0)
    scatter(didx0, pbuf0, s0)
    wait_p(pbuf1, s1)
    fi_dst(cl - 1, didx1, di1)
    w_idx(sidx1, si1)
    gather(sidx1, pbuf1, g1)
    wait_p(pbuf1, g1)
    wait_e(ebuf1, e1)
    addin(pbuf1, ebuf1)
    w_idx(didx1, di1)
    scatter(didx1, pbuf1, s1)
    wait_p(pbuf0, s0)
    wait_p(pbuf1, s1)
    plsc.subcore_barrier()
    obase = cid * NP + row0
    for r in range(RPT // CH):
        pltpu.sync_copy(acc_sp.at[pl.ds(row0 + r * CH, CH)], pbuf0)
        pltpu.sync_copy(pbuf0, s1_hbm.at[pl.ds(obase + r * CH, CH)])


@functools.partial(
    pl.kernel,
    out_type=jax.ShapeDtypeStruct((NC * NP, DOUT), _F32),
    mesh=_MESH,
    scratch_types=[
        pltpu.VMEM((CH,), jnp.int32),
        pltpu.VMEM((CH,), jnp.int32),
        pltpu.VMEM((CH,), jnp.int32),
        pltpu.VMEM((CH,), jnp.int32),
        pltpu.VMEM((CH, DOUT), _F32),
        pltpu.VMEM((CH, DOUT), _F32),
        pltpu.VMEM_SHARED((NP, DOUT), _F32),
        pltpu.SemaphoreType.DMA,
        pltpu.SemaphoreType.DMA,
        pltpu.SemaphoreType.DMA,
        pltpu.SemaphoreType.DMA,
        pltpu.SemaphoreType.DMA,
        pltpu.SemaphoreType.DMA,
        pltpu.SemaphoreType.DMA,
        pltpu.SemaphoreType.DMA,
    ],
)
def _sc_layer2(p_hbm, src_hbm, dst_hbm, s2_hbm,
               sidx0, sidx1, didx0, didx1, pbuf0, pbuf1,
               acc_sp, g0, g1, s0, s1, si0, si1, di0, di1):
    cid = lax.axis_index("c")
    sid = lax.axis_index("s")
    wid = cid * NS + sid
    _zero_rows(pbuf0, CH, DOUT)
    _zero_rows(pbuf1, CH, DOUT)
    row0 = sid * RPT
    for r in range(RPT // CH):
        pltpu.sync_copy(pbuf0, acc_sp.at[pl.ds(row0 + r * CH, CH)])
    koff = jnp.where(cid == 0, sid * CA, NS * CA + sid * CB)
    cl = jnp.where(cid == 0, CA, CB)
    kbl = cl // 2 - 1
    ebase = koff * CH
    pltpu.sync_copy(src_hbm.at[pl.ds(ebase, CH)], sidx0)
    plsc.subcore_barrier()

    def fi_src(k, buf, sem):
        pltpu.async_copy(src_hbm.at[pl.ds(ebase + k * CH, CH)], buf, sem)

    def fi_dst(k, buf, sem):
        pltpu.async_copy(dst_hbm.at[pl.ds(ebase + k * CH, CH)], buf, sem)

    def w_idx(buf, sem):
        pltpu.make_async_copy(src_hbm.at[pl.ds(0, CH)], buf, sem).wait()

    def gather(sbuf, buf, sem):
        pltpu.async_copy(p_hbm.at[sbuf], buf, sem)

    def scatter(dbuf, buf, sem):
        pltpu.async_copy(buf, acc_sp.at[dbuf], sem, add=True)

    def wait_p(buf, sem):
        pltpu.make_async_copy(p_hbm.at[pl.ds(0, CH)], buf, sem).wait()

    gather(sidx0, pbuf0, g0)
    fi_dst(0, didx0, di0)
    fi_src(1, sidx1, si1)
    pltpu.async_copy(pbuf1, acc_sp.at[sidx0], s1, add=True)

    def body(kk, carry):
        k0 = 2 * kk
        k1 = k0 + 1
        wait_p(pbuf0, g0)
        fi_src(k0 + 2, sidx0, si0)
        w_idx(didx0, di0)
        scatter(didx0, pbuf0, s0)
        wait_p(pbuf1, s1)
        fi_dst(k1, didx1, di1)
        w_idx(sidx1, si1)
        gather(sidx1, pbuf1, g1)
        wait_p(pbuf1, g1)
        fi_src(k1 + 2, sidx1, si1)
        w_idx(didx1, di1)
        scatter(didx1, pbuf1, s1)
        wait_p(pbuf0, s0)
        fi_dst(k0 + 2, didx0, di0)
        w_idx(sidx0, si0)
        gather(sidx0, pbuf0, g0)
        return carry

    lax.fori_loop(0, kbl, body, 0)
    wait_p(pbuf0, g0)
    w_idx(didx0, di0)
    scatter(didx0, pbuf0, s0)
    wait_p(pbuf1, s1)
    fi_dst(cl - 1, didx1, di1)
    w_idx(sidx1, si1)
    gather(sidx1, pbuf1, g1)
    wait_p(pbuf1, g1)
    w_idx(didx1, di1)
    scatter(didx1, pbuf1, s1)
    wait_p(pbuf0, s0)
    wait_p(pbuf1, s1)
    plsc.subcore_barrier()
    obase = cid * NP + row0
    for r in range(RPT // CH):
        pltpu.sync_copy(acc_sp.at[pl.ds(row0 + r * CH, CH)], pbuf0)
        pltpu.sync_copy(pbuf0, s2_hbm.at[pl.ds(obase + r * CH, CH)])


@functools.partial(
    pl.kernel,
    out_type=jax.ShapeDtypeStruct((EP8, 8 * EDIM), _F32),
    mesh=_MESH,
    scratch_types=[
        pltpu.VMEM((CA3, CH3), jnp.int32),
        pltpu.VMEM((CA3, CH3), jnp.int32),
        pltpu.VMEM((CH3, DIN), _F32),
        pltpu.VMEM((CH3, DIN), _F32),
        pltpu.VMEM((CH3, DIN), _F32),
        pltpu.VMEM((CH3, DIN), _F32),
        pltpu.VMEM((CH3 // 8, 8 * EDIM), _F32),
        pltpu.VMEM((CH3 // 8, 8 * EDIM), _F32),
        pltpu.SemaphoreType.DMA,
        pltpu.SemaphoreType.DMA,
        pltpu.SemaphoreType.DMA,
        pltpu.SemaphoreType.DMA,
        pltpu.SemaphoreType.DMA,
        pltpu.SemaphoreType.DMA,
    ],
)
def _sc_score(ab_hbm, src_hbm, dst_hbm, out_hbm,
              sidx, didx, u0, u1, v0, v1, o0, o1,
              gu0, gu1, gv0, gv1, so0, so1):
    cid = lax.axis_index("c")
    sid = lax.axis_index("s")
    wid = cid * NS + sid
    koff = jnp.where(cid == 0, sid * CA3, NS * CA3 + sid * CB3)
    cl = jnp.where(cid == 0, CA3, CB3)
    kbl = cl // 2 - 1
    pltpu.sync_copy(src_hbm.at[pl.ds(koff, CA3)], sidx)
    pltpu.sync_copy(dst_hbm.at[pl.ds(koff, CA3)], didx)
    ebase8 = koff * (CH3 // 8)

    def gu(k, buf, sem):
        pltpu.async_copy(ab_hbm.at[sidx.at[k]], buf, sem)

    def gv(k, buf, sem):
        pltpu.async_copy(ab_hbm.at[didx.at[k]], buf, sem)

    def ostore(k, buf, sem):
        pltpu.async_copy(
            buf, out_hbm.at[pl.ds(ebase8 + k * (CH3 // 8), CH3 // 8)], sem)

    def wait_w(buf, sem):
        pltpu.make_async_copy(ab_hbm.at[pl.ds(0, CH3)], buf, sem).wait()

    def wait_o(buf, sem):
        pltpu.make_async_copy(out_hbm.at[pl.ds(0, CH3 // 8)], buf, sem).wait()

    def combine(ob, ub, vb):
        for r in range(CH3):
            ob[r // 8, pl.ds((r % 8) * EDIM, EDIM)] = (
                ub[r, pl.ds(0, EDIM)] + vb[r, pl.ds(EDIM, EDIM)])

    gu(0, u0, gu0)
    gv(0, v0, gv0)
    gu(1, u1, gu1)
    gv(1, v1, gv1)
    wait_w(u0, gu0)
    wait_w(v0, gv0)
    combine(o0, u0, v0)
    ostore(0, o0, so0)
    gu(2, u0, gu0)
    gv(2, v0, gv0)
    wait_w(u1, gu1)
    wait_w(v1, gv1)
    combine(o1, u1, v1)
    ostore(1, o1, so1)
    gu(3, u1, gu1)
    gv(3, v1, gv1)

    def body(kk, carry):
        k0 = 2 * kk
        k1 = k0 + 1
        wait_w(u0, gu0)
        wait_w(v0, gv0)
        wait_o(o0, so0)
        combine(o0, u0, v0)
        ostore(k0, o0, so0)
        gu(k0 + 2, u0, gu0)
        gv(k0 + 2, v0, gv0)
        wait_w(u1, gu1)
        wait_w(v1, gv1)
        wait_o(o1, so1)
        combine(o1, u1, v1)
        ostore(k1, o1, so1)
        gu(k1 + 2, u1, gu1)
        gv(k1 + 2, v1, gv1)
        return carry

    lax.fori_loop(1, kbl, body, 0)
    wait_w(u0, gu0)
    wait_w(v0, gv0)
    wait_o(o0, so0)
    combine(o0, u0, v0)
    ostore(cl - 2, o0, so0)
    wait_w(u1, gu1)
    wait_w(v1, gv1)
    wait_o(o1, so1)
    combine(o1, u1, v1)
    ostore(cl - 1, o1, so1)
    wait_o(o0, so0)
    wait_o(o1, so1)


def kernel(nfeats, efeats, edge_index, Wm1, bm1, Wa1, ba1,
           Wm2, bm2, Wa2, ba2, Wp, bp):
    h0 = nfeats.reshape(N, DIN)
    h0p = jnp.pad(h0, ((0, NP - N), (0, 0)))
    dummy = N + (jnp.arange(EP - E, dtype=jnp.int32) % (NP - N))
    srcp = jnp.concatenate([edge_index[0], dummy])
    dstp = jnp.concatenate([edge_index[1], dummy])
    pad3 = (CA3 - CB3) * CH3
    src2d = jnp.concatenate(
        [srcp, jnp.full((pad3,), N, jnp.int32)]).reshape(-1, CH3)
    dst2d = jnp.concatenate(
        [dstp, jnp.full((pad3,), N, jnp.int32)]).reshape(-1, CH3)

    wm1t, wm1b = Wm1[:DIN], Wm1[DIN:]
    wa1t, wa1b = Wa1[:DIN], Wa1[DIN:]
    wp2 = Wm2[:EDIM] + 0.5 * Wm2[EDIM:]
    wq2 = 0.5 * Wm2[EDIM:]
    wa2t, wa2b = Wa2[:EDIM], Wa2[EDIM:]
    wpt = jnp.pad(Wp[:DOUT], ((0, 0), (0, 16 - NCLS)))
    wpb = jnp.pad(Wp[DOUT:], ((0, 0), (0, 16 - NCLS)))
    bpp = jnp.pad(bp, (0, 16 - NCLS)).reshape(1, 16)
    bm1r = bm1.reshape(1, EDIM)
    ba1r = ba1.reshape(1, EDIM)
    bm2r = bm2.reshape(1, DOUT)
    ba2r = ba2.reshape(1, DOUT)

    e8 = jnp.pad(efeats.reshape(E // 8, 8 * EDIM),
                 ((0, EP8 - E // 8), (0, 0)))
    wblk = jnp.kron(jnp.eye(8, dtype=_F32), wm1b)
    btile = jnp.tile(bm1, 8).reshape(1, 8 * EDIM)
    be8 = 320
    ew = pl.pallas_call(
        _edge_msg_body,
        grid=(EP8 // be8,),
        in_specs=[pl.BlockSpec((be8, 8 * EDIM), lambda i: (i, 0)),
                  pl.BlockSpec((8 * EDIM, 8 * EDIM), lambda i: (0, 0)),
                  pl.BlockSpec((1, 8 * EDIM), lambda i: (0, 0))],
        out_specs=pl.BlockSpec((be8, 8 * EDIM), lambda i: (i, 0)),
        out_shape=jax.ShapeDtypeStruct((EP8, 8 * EDIM), _F32),
    )(e8, wblk, btile)

    nb = 1280
    hc = pl.pallas_call(
        _node_msg_body,
        grid=(NP // nb,),
        in_specs=[pl.BlockSpec((nb, DIN), lambda i: (i, 0)),
                  pl.BlockSpec((DIN, EDIM), lambda i: (0, 0))],
        out_specs=pl.BlockSpec((nb, DIN), lambda i: (i, 0)),
        out_shape=jax.ShapeDtypeStruct((NP, DIN), _F32),
    )(h0p, wm1t)

    s1 = _sc_layer1(hc, ew, srcp, dstp).reshape(NC, NP, DIN)

    h1, pmat, qb = pl.pallas_call(
        _layer1_apply_body,
        grid=(NP // nb,),
        in_specs=[pl.BlockSpec((nb, DIN), lambda i: (i, 0)),
                  pl.BlockSpec((NC, nb, DIN), lambda i: (0, i, 0)),
                  pl.BlockSpec((DIN, EDIM), lambda i: (0, 0)),
                  pl.BlockSpec((EDIM, EDIM), lambda i: (0, 0)),
                  pl.BlockSpec((1, EDIM), lambda i: (0, 0)),
                  pl.BlockSpec((EDIM, DOUT), lambda i: (0, 0)),
                  pl.BlockSpec((EDIM, DOUT), lambda i: (0, 0)),
                  pl.BlockSpec((1, DOUT), lambda i: (0, 0))],
        out_specs=[pl.BlockSpec((nb, EDIM), lambda i: (i, 0)),
                   pl.BlockSpec((nb, DOUT), lambda i: (i, 0)),
                   pl.BlockSpec((nb, DOUT), lambda i: (i, 0))],
        out_shape=[jax.ShapeDtypeStruct((NP, EDIM), _F32),
                   jax.ShapeDtypeStruct((NP, DOUT), _F32),
                   jax.ShapeDtypeStruct((NP, DOUT), _F32)],
    )(h0p, s1, wa1t, wa1b, ba1r, wp2, wq2, bm2r)

    s2 = _sc_layer2(pmat, srcp, dstp).reshape(NC, NP, DOUT)

    ab = pl.pallas_call(
        _layer2_apply_body,
        grid=(NP // nb,),
        in_specs=[pl.BlockSpec((nb, EDIM), lambda i: (i, 0)),
                  pl.BlockSpec((NC, nb, DIN), lambda i: (0, i, 0)),
                  pl.BlockSpec((NC, nb, DOUT), lambda i: (0, i, 0)),
                  pl.BlockSpec((nb, DOUT), lambda i: (i, 0)),
                  pl.BlockSpec((EDIM, DOUT), lambda i: (0, 0)),
                  pl.BlockSpec((DOUT, DOUT), lambda i: (0, 0)),
                  pl.BlockSpec((1, DOUT), lambda i: (0, 0)),
                  pl.BlockSpec((DOUT, 16), lambda i: (0, 0)),
                  pl.BlockSpec((DOUT, 16), lambda i: (0, 0)),
                  pl.BlockSpec((1, 16), lambda i: (0, 0))],
        out_specs=pl.BlockSpec((nb, DIN), lambda i: (i, 0)),
        out_shape=jax.ShapeDtypeStruct((NP, DIN), _F32),
    )(h1, s1, s2, qb, wa2t, wa2b, ba2r, wpt, wpb, bpp)

    score = _sc_score(ab, src2d, dst2d).reshape(EP, EDIM)
    return score[:E, :NCLS]


# speedup vs baseline: 3.2376x; 1.1113x over previous
import functools
import jax
import jax.numpy as jnp
from jax import lax
from jax.experimental import pallas as pl
from jax.experimental.pallas import tpu as pltpu
from jax.experimental.pallas import tpu_sc as plsc

N = 10000
E = 320000
DIN = 128
EDIM = 16
DOUT = 128
NCLS = 11

NC = 2
NS = 16
NW = NC * NS
CH = 64
CHUNKS = 160
CH3 = 64
CHUNKS3 = 160
EPT = CH * CHUNKS
EP = EPT * NW
NP = 10240
RPT = NP // NS
KB = CHUNKS // 2 - 1
KB3 = CHUNKS3 // 2 - 1
CA = 160
CB = 160
CA3 = 80
CB3 = 80
EP8 = EP // 8

_MESH = plsc.VectorSubcoreMesh(core_axis_name="c", subcore_axis_name="s")
_F32 = jnp.float32
_HIGH = lax.Precision.HIGHEST


def _dot(a, b):
    return jnp.dot(a, b, preferred_element_type=_F32, precision=_HIGH)


def _edge_msg_body(e_ref, w_ref, b_ref, o_ref):
    o_ref[...] = _dot(e_ref[...], w_ref[...]) + b_ref[...]


def _node_msg_body(h_ref, w_ref, o_ref):
    nb = h_ref.shape[0]
    hw = _dot(h_ref[...], w_ref[...])
    col = lax.broadcasted_iota(jnp.int32, (nb, DIN), 1)
    wide = jnp.concatenate(
        [hw, jnp.zeros((nb, DIN - EDIM), _F32)], axis=1)
    o_ref[...] = jnp.where(col == EDIM, 1.0, wide)


def _layer1_apply_body(h0_ref, s1_ref, wa1t_ref, wa1b_ref, ba1_ref,
                       wp2_ref, wq2_ref, bm2_ref, h1_ref, p_ref, qb_ref):
    acc = s1_ref[0] + s1_ref[1]
    cnt = acc[:, EDIM]
    inv = 1.0 / jnp.maximum(cnt, 1.0)
    hn1 = acc[:, :EDIM] * inv[:, None]
    h1 = jax.nn.relu(_dot(h0_ref[...], wa1t_ref[...]) +
                     _dot(hn1, wa1b_ref[...]) + ba1_ref[...])
    h1_ref[...] = h1
    p_ref[...] = _dot(h1, wp2_ref[...])
    qb_ref[...] = _dot(h1, wq2_ref[...]) + bm2_ref[...]


def _layer2_apply_body(h1_ref, s1_ref, s2_ref, qb_ref, wa2t_ref, wa2b_ref,
                       ba2_ref, wpt_ref, wpb_ref, bp_ref, ab_ref):
    nb = h1_ref.shape[0]
    cnt = s1_ref[0][:, EDIM] + s1_ref[1][:, EDIM]
    inv = 1.0 / jnp.maximum(cnt, 1.0)
    gate = jnp.minimum(cnt, 1.0)
    s2t = s2_ref[0] + s2_ref[1]
    hn2 = s2t * inv[:, None] + qb_ref[...] * gate[:, None]
    h2 = jax.nn.relu(_dot(h1_ref[...], wa2t_ref[...]) +
                     _dot(hn2, wa2b_ref[...]) + ba2_ref[...])
    a = _dot(h2, wpt_ref[...]) + bp_ref[...]
    b = _dot(h2, wpb_ref[...])
    ab_ref[...] = jnp.concatenate(
        [a, b, jnp.zeros((nb, DIN - 2 * EDIM), _F32)], axis=1)


def _zero_rows(buf, rows, width):
    zero = jnp.zeros((16,), _F32)
    for r in range(rows):
        for c in range(width // 16):
            buf[r, pl.ds(c * 16, 16)] = zero


@functools.partial(
    pl.kernel,
    out_type=jax.ShapeDtypeStruct((NC * NP, DIN), _F32),
    mesh=_MESH,
    scratch_types=[
        pltpu.VMEM((CH,), jnp.int32),
        pltpu.VMEM((CH,), jnp.int32),
        pltpu.VMEM((CH,), jnp.int32),
        pltpu.VMEM((CH,), jnp.int32),
        pltpu.VMEM((CH, DIN), _F32),
        pltpu.VMEM((CH, DIN), _F32),
        pltpu.VMEM((CH // 8, 8 * EDIM), _F32),
        pltpu.VMEM((CH // 8, 8 * EDIM), _F32),
        pltpu.VMEM_SHARED((NP, DIN), _F32),
        pltpu.SemaphoreType.DMA,
        pltpu.SemaphoreType.DMA,
        pltpu.SemaphoreType.DMA,
        pltpu.SemaphoreType.DMA,
        pltpu.SemaphoreType.DMA,
        pltpu.SemaphoreType.DMA,
        pltpu.SemaphoreType.DMA,
        pltpu.SemaphoreType.DMA,
        pltpu.SemaphoreType.DMA,
        pltpu.SemaphoreType.DMA,
    ],
)
def _sc_layer1(hc_hbm, ew_hbm, src_hbm, dst_hbm, s1_hbm,
               sidx0, sidx1, didx0, didx1, pbuf0, pbuf1, ebuf0, ebuf1,
               acc_sp, g0, g1, s0, s1, e0, e1, si0, si1, di0, di1):
    cid = lax.axis_index("c")
    sid = lax.axis_index("s")
    wid = cid * NS + sid
    _zero_rows(pbuf0, CH, DIN)
    _zero_rows(pbuf1, CH, DIN)
    row0 = sid * RPT
    for r in range(RPT // CH):
        pltpu.sync_copy(pbuf0, acc_sp.at[pl.ds(row0 + r * CH, CH)])
    koff = jnp.where(cid == 0, sid * CA, NS * CA + sid * CB)
    cl = jnp.where(cid == 0, CA, CB)
    kbl = cl // 2 - 1
    ebase = koff * CH
    ebase8 = koff * 8
    pltpu.sync_copy(src_hbm.at[pl.ds(ebase, CH)], sidx0)
    plsc.subcore_barrier()

    def fi_src(k, buf, sem):
        pltpu.async_copy(src_hbm.at[pl.ds(ebase + k * CH, CH)], buf, sem)

    def fi_dst(k, buf, sem):
        pltpu.async_copy(dst_hbm.at[pl.ds(ebase + k * CH, CH)], buf, sem)

    def w_idx(buf, sem):
        pltpu.make_async_copy(src_hbm.at[pl.ds(0, CH)], buf, sem).wait()

    def gather(sbuf, buf, sem):
        pltpu.async_copy(hc_hbm.at[sbuf], buf, sem)

    def scatter(dbuf, buf, sem):
        pltpu.async_copy(buf, acc_sp.at[dbuf], sem, add=True)

    def eload(k, buf, sem):
        pltpu.async_copy(ew_hbm.at[pl.ds(ebase8 + k * (CH // 8), CH // 8)],
                         buf, sem)

    def wait_p(buf, sem):
        pltpu.make_async_copy(hc_hbm.at[pl.ds(0, CH)], buf, sem).wait()

    def wait_e(buf, sem):
        pltpu.make_async_copy(ew_hbm.at[pl.ds(0, CH // 8)], buf, sem).wait()

    def addin(pbuf, ebuf):
        for r in range(CH):
            pbuf[r, pl.ds(0, EDIM)] = (
                pbuf[r, pl.ds(0, EDIM)]
                + ebuf[r // 8, pl.ds((r % 8) * EDIM, EDIM)])

    gather(sidx0, pbuf0, g0)
    fi_dst(0, didx0, di0)
    fi_src(1, sidx1, si1)
    eload(0, ebuf0, e0)
    eload(1, ebuf1, e1)
    pltpu.async_copy(pbuf1, acc_sp.at[sidx0], s1, add=True)

    def body(kk, carry):
        k0 = 2 * kk
        k1 = k0 + 1
        wait_p(pbuf0, g0)
        fi_src(k0 + 2, sidx0, si0)
        wait_e(ebuf0, e0)
        addin(pbuf0, ebuf0)
        eload(k0 + 2, ebuf0, e0)
        w_idx(didx0, di0)
        scatter(didx0, pbuf0, s0)
        wait_p(pbuf1, s1)
        fi_dst(k1, didx1, di1)
        w_idx(sidx1, si1)
        gather(sidx1, pbuf1, g1)
        wait_p(pbuf1, g1)
        fi_src(k1 + 2, sidx1, si1)
        wait_e(ebuf1, e1)
        addin(pbuf1, ebuf1)
        eload(k1 + 2, ebuf1, e1)
        w_idx(didx1, di1)
        scatter(didx1, pbuf1, s1)
        wait_p(pbuf0, s0)
        fi_dst(k0 + 2, didx0, di0)
        w_idx(sidx0, si0)
        gather(sidx0, pbuf0, g0)
        return carry

    lax.fori_loop(0, kbl, body, 0)
    wait_p(pbuf0, g0)
    wait_e(ebuf0, e0)
    addin(pbuf0, ebuf0)
    w_idx(didx0, di0)
    scatter(didx0, pbuf0, s0)
    wait_p(pbuf1, s1)
    fi_dst(cl - 1, didx1, di1)
    w_idx(sidx1, si1)
    gather(sidx1, pbuf1, g1)
    wait_p(pbuf1, g1)
    wait_e(ebuf1, e1)
    addin(pbuf1, ebuf1)
    w_idx(didx1, di1)
    scatter(didx1, pbuf1, s1)
    wait_p(pbuf0, s0)
    wait_p(pbuf1, s1)
    plsc.subcore_barrier()
    obase = cid * NP + row0
    for r in range(RPT // CH):
        pltpu.sync_copy(acc_sp.at[pl.ds(row0 + r * CH, CH)], pbuf0)
        pltpu.sync_copy(pbuf0, s1_hbm.at[pl.ds(obase + r * CH, CH)])


@functools.partial(
    pl.kernel,
    out_type=jax.ShapeDtypeStruct((NC * NP, DOUT), _F32),
    mesh=_MESH,
    scratch_types=[
        pltpu.VMEM((CH,), jnp.int32),
        pltpu.VMEM((CH,), jnp.int32),
        pltpu.VMEM((CH,), jnp.int32),
        pltpu.VMEM((CH,), jnp.int32),
        pltpu.VMEM((CH, DOUT), _F32),
        pltpu.VMEM((CH, DOUT), _F32),
        pltpu.VMEM_SHARED((NP, DOUT), _F32),
        pltpu.SemaphoreType.DMA,
        pltpu.SemaphoreType.DMA,
        pltpu.SemaphoreType.DMA,
        pltpu.SemaphoreType.DMA,
        pltpu.SemaphoreType.DMA,
        pltpu.SemaphoreType.DMA,
        pltpu.SemaphoreType.DMA,
        pltpu.SemaphoreType.DMA,
    ],
)
def _sc_layer2(p_hbm, src_hbm, dst_hbm, s2_hbm,
               sidx0, sidx1, didx0, didx1, pbuf0, pbuf1,
               acc_sp, g0, g1, s0, s1, si0, si1, di0, di1):
    cid = lax.axis_index("c")
    sid = lax.axis_index("s")
    wid = cid * NS + sid
    _zero_rows(pbuf0, CH, DOUT)
    _zero_rows(pbuf1, CH, DOUT)
    row0 = sid * RPT
    for r in range(RPT // CH):
        pltpu.sync_copy(pbuf0, acc_sp.at[pl.ds(row0 + r * CH, CH)])
    koff = jnp.where(cid == 0, sid * CA, NS * CA + sid * CB)
    cl = jnp.where(cid == 0, CA, CB)
    kbl = cl // 2 - 1
    ebase = koff * CH
    pltpu.sync_copy(src_hbm.at[pl.ds(ebase, CH)], sidx0)
    plsc.subcore_barrier()

    def fi_src(k, buf, sem):
        pltpu.async_copy(src_hbm.at[pl.ds(ebase + k * CH, CH)], buf, sem)

    def fi_dst(k, buf, sem):
        pltpu.async_copy(dst_hbm.at[pl.ds(ebase + k * CH, CH)], buf, sem)

    def w_idx(buf, sem):
        pltpu.make_async_copy(src_hbm.at[pl.ds(0, CH)], buf, sem).wait()

    def gather(sbuf, buf, sem):
        pltpu.async_copy(p_hbm.at[sbuf], buf, sem)

    def scatter(dbuf, buf, sem):
        pltpu.async_copy(buf, acc_sp.at[dbuf], sem, add=True)

    def wait_p(buf, sem):
        pltpu.make_async_copy(p_hbm.at[pl.ds(0, CH)], buf, sem).wait()

    gather(sidx0, pbuf0, g0)
    fi_dst(0, didx0, di0)
    fi_src(1, sidx1, si1)
    pltpu.async_copy(pbuf1, acc_sp.at[sidx0], s1, add=True)

    def body(kk, carry):
        k0 = 2 * kk
        k1 = k0 + 1
        wait_p(pbuf0, g0)
        fi_src(k0 + 2, sidx0, si0)
        w_idx(didx0, di0)
        scatter(didx0, pbuf0, s0)
        wait_p(pbuf1, s1)
        fi_dst(k1, didx1, di1)
        w_idx(sidx1, si1)
        gather(sidx1, pbuf1, g1)
        wait_p(pbuf1, g1)
        fi_src(k1 + 2, sidx1, si1)
        w_idx(didx1, di1)
        scatter(didx1, pbuf1, s1)
        wait_p(pbuf0, s0)
        fi_dst(k0 + 2, didx0, di0)
        w_idx(sidx0, si0)
        gather(sidx0, pbuf0, g0)
        return carry

    lax.fori_loop(0, kbl, body, 0)
    wait_p(pbuf0, g0)
    w_idx(didx0, di0)
    scatter(didx0, pbuf0, s0)
    wait_p(pbuf1, s1)
    fi_dst(cl - 1, didx1, di1)
    w_idx(sidx1, si1)
    gather(sidx1, pbuf1, g1)
    wait_p(pbuf1, g1)
    w_idx(didx1, di1)
    scatter(didx1, pbuf1, s1)
    wait_p(pbuf0, s0)
    wait_p(pbuf1, s1)
    plsc.subcore_barrier()
    obase = cid * NP + row0
    for r in range(RPT // CH):
        pltpu.sync_copy(acc_sp.at[pl.ds(row0 + r * CH, CH)], pbuf0)
        pltpu.sync_copy(pbuf0, s2_hbm.at[pl.ds(obase + r * CH, CH)])


@functools.partial(
    pl.kernel,
    out_type=jax.ShapeDtypeStruct((EP, EDIM), _F32),
    mesh=_MESH,
    scratch_types=[
        pltpu.VMEM((EPT,), jnp.int32),
        pltpu.VMEM((EPT,), jnp.int32),
        pltpu.VMEM((CH3, DIN), _F32),
        pltpu.VMEM((CH3, DIN), _F32),
        pltpu.VMEM((CH3, DIN), _F32),
        pltpu.VMEM((CH3, DIN), _F32),
        pltpu.VMEM((CH3, EDIM), _F32),
        pltpu.VMEM((CH3, EDIM), _F32),
        pltpu.SemaphoreType.DMA,
        pltpu.SemaphoreType.DMA,
        pltpu.SemaphoreType.DMA,
        pltpu.SemaphoreType.DMA,
        pltpu.SemaphoreType.DMA,
        pltpu.SemaphoreType.DMA,
    ],
)
def _sc_score(ab_hbm, src_hbm, dst_hbm, out_hbm,
              sidx, didx, u0, u1, v0, v1, o0, o1,
              gu0, gu1, gv0, gv1, so0, so1):
    cid = lax.axis_index("c")
    sid = lax.axis_index("s")
    wid = cid * NS + sid
    ebase = wid * EPT
    pltpu.sync_copy(src_hbm.at[pl.ds(ebase, EPT)], sidx)
    pltpu.sync_copy(dst_hbm.at[pl.ds(ebase, EPT)], didx)

    def gu(k, buf, sem):
        pltpu.async_copy(ab_hbm.at[sidx.at[pl.ds(k * CH3, CH3)]], buf, sem)

    def gv(k, buf, sem):
        pltpu.async_copy(ab_hbm.at[didx.at[pl.ds(k * CH3, CH3)]], buf, sem)

    def ostore(k, buf, sem):
        pltpu.async_copy(buf, out_hbm.at[pl.ds(ebase + k * CH3, CH3)], sem)

    def wait_w(buf, sem):
        pltpu.make_async_copy(ab_hbm.at[pl.ds(0, CH3)], buf, sem).wait()

    def wait_o(buf, sem):
        pltpu.make_async_copy(out_hbm.at[pl.ds(0, CH3)], buf, sem).wait()

    def combine(ob, ub, vb):
        for r in range(CH3):
            ob[r, :] = ub[r, pl.ds(0, EDIM)] + vb[r, pl.ds(EDIM, EDIM)]

    gu(0, u0, gu0)
    gv(0, v0, gv0)
    gu(1, u1, gu1)
    gv(1, v1, gv1)
    wait_w(u0, gu0)
    wait_w(v0, gv0)
    combine(o0, u0, v0)
    ostore(0, o0, so0)
    gu(2, u0, gu0)
    gv(2, v0, gv0)
    wait_w(u1, gu1)
    wait_w(v1, gv1)
    combine(o1, u1, v1)
    ostore(1, o1, so1)
    gu(3, u1, gu1)
    gv(3, v1, gv1)

    def body(kk, carry):
        k0 = 2 * kk
        k1 = k0 + 1
        wait_w(u0, gu0)
        wait_w(v0, gv0)
        wait_o(o0, so0)
        combine(o0, u0, v0)
        ostore(k0, o0, so0)
        gu(k0 + 2, u0, gu0)
        gv(k0 + 2, v0, gv0)
        wait_w(u1, gu1)
        wait_w(v1, gv1)
        wait_o(o1, so1)
        combine(o1, u1, v1)
        ostore(k1, o1, so1)
        gu(k1 + 2, u1, gu1)
        gv(k1 + 2, v1, gv1)
        return carry

    lax.fori_loop(1, CHUNKS3 // 2 - 1, body, 0)
    wait_w(u0, gu0)
    wait_w(v0, gv0)
    wait_o(o0, so0)
    combine(o0, u0, v0)
    ostore(CHUNKS3 - 2, o0, so0)
    wait_w(u1, gu1)
    wait_w(v1, gv1)
    wait_o(o1, so1)
    combine(o1, u1, v1)
    ostore(CHUNKS3 - 1, o1, so1)
    wait_o(o0, so0)
    wait_o(o1, so1)


def kernel(nfeats, efeats, edge_index, Wm1, bm1, Wa1, ba1,
           Wm2, bm2, Wa2, ba2, Wp, bp):
    h0 = nfeats.reshape(N, DIN)
    h0p = jnp.pad(h0, ((0, NP - N), (0, 0)))
    dummy = N + (jnp.arange(EP - E, dtype=jnp.int32) % (NP - N))
    srcp = jnp.concatenate([edge_index[0], dummy])
    dstp = jnp.concatenate([edge_index[1], dummy])
    pad3 = (CA3 - CB3) * CH3
    src2d = jnp.concatenate(
        [srcp, jnp.full((pad3,), N, jnp.int32)]).reshape(-1, CH3)
    dst2d = jnp.concatenate(
        [dstp, jnp.full((pad3,), N, jnp.int32)]).reshape(-1, CH3)

    wm1t, wm1b = Wm1[:DIN], Wm1[DIN:]
    wa1t, wa1b = Wa1[:DIN], Wa1[DIN:]
    wp2 = Wm2[:EDIM] + 0.5 * Wm2[EDIM:]
    wq2 = 0.5 * Wm2[EDIM:]
    wa2t, wa2b = Wa2[:EDIM], Wa2[EDIM:]
    wpt = jnp.pad(Wp[:DOUT], ((0, 0), (0, 16 - NCLS)))
    wpb = jnp.pad(Wp[DOUT:], ((0, 0), (0, 16 - NCLS)))
    bpp = jnp.pad(bp, (0, 16 - NCLS)).reshape(1, 16)
    bm1r = bm1.reshape(1, EDIM)
    ba1r = ba1.reshape(1, EDIM)
    bm2r = bm2.reshape(1, DOUT)
    ba2r = ba2.reshape(1, DOUT)

    e8 = jnp.pad(efeats.reshape(E // 8, 8 * EDIM),
                 ((0, EP8 - E // 8), (0, 0)))
    wblk = jnp.kron(jnp.eye(8, dtype=_F32), wm1b)
    btile = jnp.tile(bm1, 8).reshape(1, 8 * EDIM)
    be8 = 2560
    ew = pl.pallas_call(
        _edge_msg_body,
        grid=(EP8 // be8,),
        in_specs=[pl.BlockSpec((be8, 8 * EDIM), lambda i: (i, 0)),
                  pl.BlockSpec((8 * EDIM, 8 * EDIM), lambda i: (0, 0)),
                  pl.BlockSpec((1, 8 * EDIM), lambda i: (0, 0))],
        out_specs=pl.BlockSpec((be8, 8 * EDIM), lambda i: (i, 0)),
        out_shape=jax.ShapeDtypeStruct((EP8, 8 * EDIM), _F32),
    )(e8, wblk, btile)

    nb = 1280
    hc = pl.pallas_call(
        _node_msg_body,
        grid=(NP // nb,),
        in_specs=[pl.BlockSpec((nb, DIN), lambda i: (i, 0)),
                  pl.BlockSpec((DIN, EDIM), lambda i: (0, 0))],
        out_specs=pl.BlockSpec((nb, DIN), lambda i: (i, 0)),
        out_shape=jax.ShapeDtypeStruct((NP, DIN), _F32),
    )(h0p, wm1t)

    s1 = _sc_layer1(hc, ew, srcp, dstp).reshape(NC, NP, DIN)

    h1, pmat, qb = pl.pallas_call(
        _layer1_apply_body,
        grid=(NP // nb,),
        in_specs=[pl.BlockSpec((nb, DIN), lambda i: (i, 0)),
                  pl.BlockSpec((NC, nb, DIN), lambda i: (0, i, 0)),
                  pl.BlockSpec((DIN, EDIM), lambda i: (0, 0)),
                  pl.BlockSpec((EDIM, EDIM), lambda i: (0, 0)),
                  pl.BlockSpec((1, EDIM), lambda i: (0, 0)),
                  pl.BlockSpec((EDIM, DOUT), lambda i: (0, 0)),
                  pl.BlockSpec((EDIM, DOUT), lambda i: (0, 0)),
                  pl.BlockSpec((1, DOUT), lambda i: (0, 0))],
        out_specs=[pl.BlockSpec((nb, EDIM), lambda i: (i, 0)),
                   pl.BlockSpec((nb, DOUT), lambda i: (i, 0)),
                   pl.BlockSpec((nb, DOUT), lambda i: (i, 0))],
        out_shape=[jax.ShapeDtypeStruct((NP, EDIM), _F32),
                   jax.ShapeDtypeStruct((NP, DOUT), _F32),
                   jax.ShapeDtypeStruct((NP, DOUT), _F32)],
    )(h0p, s1, wa1t, wa1b, ba1r, wp2, wq2, bm2r)

    s2 = _sc_layer2(pmat, srcp, dstp).reshape(NC, NP, DOUT)

    ab = pl.pallas_call(
        _layer2_apply_body,
        grid=(NP // nb,),
        in_specs=[pl.BlockSpec((nb, EDIM), lambda i: (i, 0)),
                  pl.BlockSpec((NC, nb, DIN), lambda i: (0, i, 0)),
                  pl.BlockSpec((NC, nb, DOUT), lambda i: (0, i, 0)),
                  pl.BlockSpec((nb, DOUT), lambda i: (i, 0)),
                  pl.BlockSpec((EDIM, DOUT), lambda i: (0, 0)),
                  pl.BlockSpec((DOUT, DOUT), lambda i: (0, 0)),
                  pl.BlockSpec((1, DOUT), lambda i: (0, 0)),
                  pl.BlockSpec((DOUT, 16), lambda i: (0, 0)),
                  pl.BlockSpec((DOUT, 16), lambda i: (0, 0)),
                  pl.BlockSpec((1, 16), lambda i: (0, 0))],
        out_specs=pl.BlockSpec((nb, DIN), lambda i: (i, 0)),
        out_shape=jax.ShapeDtypeStruct((NP, DIN), _F32),
    )(h1, s1, s2, qb, wa2t, wa2b, ba2r, wpt, wpb, bpp)

    score = _sc_score(ab, srcp, dstp)
    return score[:E, :NCLS]


# speedup vs baseline: 3.5475x; 1.0957x over previous
import functools
import jax
import jax.numpy as jnp
from jax import lax
from jax.experimental import pallas as pl
from jax.experimental.pallas import tpu as pltpu
from jax.experimental.pallas import tpu_sc as plsc

N = 10000
E = 320000
DIN = 128
EDIM = 16
DOUT = 128
NCLS = 11

NC = 2
NS = 16
NW = NC * NS
CH = 128
CHUNKS = 80
CH3 = 64
CHUNKS3 = 160
EPT = CH * CHUNKS
EP = EPT * NW
NP = 10240
RPT = NP // NS
KB = CHUNKS // 2 - 1
KB3 = CHUNKS3 // 2 - 1
CA = 80
CB = 80
CA3 = 80
CB3 = 80
EP8 = EP // 8

_MESH = plsc.VectorSubcoreMesh(core_axis_name="c", subcore_axis_name="s")
_F32 = jnp.float32
_HIGH = lax.Precision.HIGHEST


def _dot(a, b):
    return jnp.dot(a, b, preferred_element_type=_F32, precision=_HIGH)


def _edge_msg_body(e_ref, w_ref, b_ref, o_ref):
    o_ref[...] = _dot(e_ref[...], w_ref[...]) + b_ref[...]


def _node_msg_body(h_ref, w_ref, o_ref):
    nb = h_ref.shape[0]
    hw = _dot(h_ref[...], w_ref[...])
    col = lax.broadcasted_iota(jnp.int32, (nb, DIN), 1)
    wide = jnp.concatenate(
        [hw, jnp.zeros((nb, DIN - EDIM), _F32)], axis=1)
    o_ref[...] = jnp.where(col == EDIM, 1.0, wide)


def _layer1_apply_body(h0_ref, s1_ref, wa1t_ref, wa1b_ref, ba1_ref,
                       wp2_ref, wq2_ref, bm2_ref, h1_ref, p_ref, qb_ref):
    acc = s1_ref[0] + s1_ref[1]
    cnt = acc[:, EDIM]
    inv = 1.0 / jnp.maximum(cnt, 1.0)
    hn1 = acc[:, :EDIM] * inv[:, None]
    h1 = jax.nn.relu(_dot(h0_ref[...], wa1t_ref[...]) +
                     _dot(hn1, wa1b_ref[...]) + ba1_ref[...])
    h1_ref[...] = h1
    p_ref[...] = _dot(h1, wp2_ref[...])
    qb_ref[...] = _dot(h1, wq2_ref[...]) + bm2_ref[...]


def _layer2_apply_body(h1_ref, s1_ref, s2_ref, qb_ref, wa2t_ref, wa2b_ref,
                       ba2_ref, wpt_ref, wpb_ref, bp_ref, ab_ref):
    nb = h1_ref.shape[0]
    cnt = s1_ref[0][:, EDIM] + s1_ref[1][:, EDIM]
    inv = 1.0 / jnp.maximum(cnt, 1.0)
    gate = jnp.minimum(cnt, 1.0)
    s2t = s2_ref[0] + s2_ref[1]
    hn2 = s2t * inv[:, None] + qb_ref[...] * gate[:, None]
    h2 = jax.nn.relu(_dot(h1_ref[...], wa2t_ref[...]) +
                     _dot(hn2, wa2b_ref[...]) + ba2_ref[...])
    a = _dot(h2, wpt_ref[...]) + bp_ref[...]
    b = _dot(h2, wpb_ref[...])
    ab_ref[...] = jnp.concatenate(
        [a, b, jnp.zeros((nb, DIN - 2 * EDIM), _F32)], axis=1)


def _zero_rows(buf, rows, width):
    zero = jnp.zeros((16,), _F32)
    for r in range(rows):
        for c in range(width // 16):
            buf[r, pl.ds(c * 16, 16)] = zero


@functools.partial(
    pl.kernel,
    out_type=jax.ShapeDtypeStruct((NC * NP, DIN), _F32),
    mesh=_MESH,
    scratch_types=[
        pltpu.VMEM((CH,), jnp.int32),
        pltpu.VMEM((CH,), jnp.int32),
        pltpu.VMEM((CH,), jnp.int32),
        pltpu.VMEM((CH,), jnp.int32),
        pltpu.VMEM((CH, DIN), _F32),
        pltpu.VMEM((CH, DIN), _F32),
        pltpu.VMEM((CH // 8, 8 * EDIM), _F32),
        pltpu.VMEM((CH // 8, 8 * EDIM), _F32),
        pltpu.VMEM_SHARED((NP, DIN), _F32),
        pltpu.SemaphoreType.DMA,
        pltpu.SemaphoreType.DMA,
        pltpu.SemaphoreType.DMA,
        pltpu.SemaphoreType.DMA,
        pltpu.SemaphoreType.DMA,
        pltpu.SemaphoreType.DMA,
        pltpu.SemaphoreType.DMA,
        pltpu.SemaphoreType.DMA,
        pltpu.SemaphoreType.DMA,
        pltpu.SemaphoreType.DMA,
    ],
)
def _sc_layer1(hc_hbm, ew_hbm, src_hbm, dst_hbm, s1_hbm,
               sidx0, sidx1, didx0, didx1, pbuf0, pbuf1, ebuf0, ebuf1,
               acc_sp, g0, g1, s0, s1, e0, e1, si0, si1, di0, di1):
    cid = lax.axis_index("c")
    sid = lax.axis_index("s")
    wid = cid * NS + sid
    _zero_rows(pbuf0, CH, DIN)
    _zero_rows(pbuf1, CH, DIN)
    row0 = sid * RPT
    for r in range(RPT // CH):
        pltpu.sync_copy(pbuf0, acc_sp.at[pl.ds(row0 + r * CH, CH)])
    koff = jnp.where(cid == 0, sid * CA, NS * CA + sid * CB)
    cl = jnp.where(cid == 0, CA, CB)
    kbl = cl // 2 - 1
    ebase = koff * CH
    ebase8 = koff * 8
    pltpu.sync_copy(src_hbm.at[pl.ds(ebase, CH)], sidx0)
    plsc.subcore_barrier()

    def fi_src(k, buf, sem):
        pltpu.async_copy(src_hbm.at[pl.ds(ebase + k * CH, CH)], buf, sem)

    def fi_dst(k, buf, sem):
        pltpu.async_copy(dst_hbm.at[pl.ds(ebase + k * CH, CH)], buf, sem)

    def w_idx(buf, sem):
        pltpu.make_async_copy(src_hbm.at[pl.ds(0, CH)], buf, sem).wait()

    def gather(sbuf, buf, sem):
        pltpu.async_copy(hc_hbm.at[sbuf], buf, sem)

    def scatter(dbuf, buf, sem):
        pltpu.async_copy(buf, acc_sp.at[dbuf], sem, add=True)

    def eload(k, buf, sem):
        pltpu.async_copy(ew_hbm.at[pl.ds(ebase8 + k * (CH // 8), CH // 8)],
                         buf, sem)

    def wait_p(buf, sem):
        pltpu.make_async_copy(hc_hbm.at[pl.ds(0, CH)], buf, sem).wait()

    def wait_e(buf, sem):
        pltpu.make_async_copy(ew_hbm.at[pl.ds(0, CH // 8)], buf, sem).wait()

    def addin(pbuf, ebuf):
        for r in range(CH):
            pbuf[r, pl.ds(0, EDIM)] = (
                pbuf[r, pl.ds(0, EDIM)]
                + ebuf[r // 8, pl.ds((r % 8) * EDIM, EDIM)])

    gather(sidx0, pbuf0, g0)
    fi_dst(0, didx0, di0)
    fi_src(1, sidx1, si1)
    eload(0, ebuf0, e0)
    eload(1, ebuf1, e1)
    pltpu.async_copy(pbuf1, acc_sp.at[sidx0], s1, add=True)

    def body(kk, carry):
        k0 = 2 * kk
        k1 = k0 + 1
        wait_p(pbuf0, g0)
        fi_src(k0 + 2, sidx0, si0)
        wait_e(ebuf0, e0)
        addin(pbuf0, ebuf0)
        eload(k0 + 2, ebuf0, e0)
        w_idx(didx0, di0)
        scatter(didx0, pbuf0, s0)
        wait_p(pbuf1, s1)
        fi_dst(k1, didx1, di1)
        w_idx(sidx1, si1)
        gather(sidx1, pbuf1, g1)
        wait_p(pbuf1, g1)
        fi_src(k1 + 2, sidx1, si1)
        wait_e(ebuf1, e1)
        addin(pbuf1, ebuf1)
        eload(k1 + 2, ebuf1, e1)
        w_idx(didx1, di1)
        scatter(didx1, pbuf1, s1)
        wait_p(pbuf0, s0)
        fi_dst(k0 + 2, didx0, di0)
        w_idx(sidx0, si0)
        gather(sidx0, pbuf0, g0)
        return carry

    lax.fori_loop(0, kbl, body, 0)
    wait_p(pbuf0, g0)
    wait_e(ebuf0, e0)
    addin(pbuf0, ebuf0)
    w_idx(didx0, di0)
    scatter(didx0, pbuf0, s0)
    wait_p(pbuf1, s1)
    fi_dst(cl - 1, didx1, di1)
    w_idx(sidx1, si1)
    gather(sidx1, pbuf1, g1)
    wait_p(pbuf1, g1)
    wait_e(ebuf1, e1)
    addin(pbuf1, ebuf1)
    w_idx(didx1, di1)
    scatter(didx1, pbuf1, s1)
    wait_p(pbuf0, s0)
    wait_p(pbuf1, s1)
    plsc.subcore_barrier()
    obase = cid * NP + row0
    for r in range(RPT // CH):
        pltpu.sync_copy(acc_sp.at[pl.ds(row0 + r * CH, CH)], pbuf0)
        pltpu.sync_copy(pbuf0, s1_hbm.at[pl.ds(obase + r * CH, CH)])


@functools.partial(
    pl.kernel,
    out_type=jax.ShapeDtypeStruct((NC * NP, DOUT), _F32),
    mesh=_MESH,
    scratch_types=[
        pltpu.VMEM((CH,), jnp.int32),
        pltpu.VMEM((CH,), jnp.int32),
        pltpu.VMEM((CH,), jnp.int32),
        pltpu.VMEM((CH,), jnp.int32),
        pltpu.VMEM((CH, DOUT), _F32),
        pltpu.VMEM((CH, DOUT), _F32),
        pltpu.VMEM_SHARED((NP, DOUT), _F32),
        pltpu.SemaphoreType.DMA,
        pltpu.SemaphoreType.DMA,
        pltpu.SemaphoreType.DMA,
        pltpu.SemaphoreType.DMA,
        pltpu.SemaphoreType.DMA,
        pltpu.SemaphoreType.DMA,
        pltpu.SemaphoreType.DMA,
        pltpu.SemaphoreType.DMA,
    ],
)
def _sc_layer2(p_hbm, src_hbm, dst_hbm, s2_hbm,
               sidx0, sidx1, didx0, didx1, pbuf0, pbuf1,
               acc_sp, g0, g1, s0, s1, si0, si1, di0, di1):
    cid = lax.axis_index("c")
    sid = lax.axis_index("s")
    wid = cid * NS + sid
    _zero_rows(pbuf0, CH, DOUT)
    _zero_rows(pbuf1, CH, DOUT)
    row0 = sid * RPT
    for r in range(RPT // CH):
        pltpu.sync_copy(pbuf0, acc_sp.at[pl.ds(row0 + r * CH, CH)])
    koff = jnp.where(cid == 0, sid * CA, NS * CA + sid * CB)
    cl = jnp.where(cid == 0, CA, CB)
    kbl = cl // 2 - 1
    ebase = koff * CH
    pltpu.sync_copy(src_hbm.at[pl.ds(ebase, CH)], sidx0)
    plsc.subcore_barrier()

    def fi_src(k, buf, sem):
        pltpu.async_copy(src_hbm.at[pl.ds(ebase + k * CH, CH)], buf, sem)

    def fi_dst(k, buf, sem):
        pltpu.async_copy(dst_hbm.at[pl.ds(ebase + k * CH, CH)], buf, sem)

    def w_idx(buf, sem):
        pltpu.make_async_copy(src_hbm.at[pl.ds(0, CH)], buf, sem).wait()

    def gather(sbuf, buf, sem):
        pltpu.async_copy(p_hbm.at[sbuf], buf, sem)

    def scatter(dbuf, buf, sem):
        pltpu.async_copy(buf, acc_sp.at[dbuf], sem, add=True)

    def wait_p(buf, sem):
        pltpu.make_async_copy(p_hbm.at[pl.ds(0, CH)], buf, sem).wait()

    gather(sidx0, pbuf0, g0)
    fi_dst(0, didx0, di0)
    fi_src(1, sidx1, si1)
    pltpu.async_copy(pbuf1, acc_sp.at[sidx0], s1, add=True)

    def body(kk, carry):
        k0 = 2 * kk
        k1 = k0 + 1
        wait_p(pbuf0, g0)
        fi_src(k0 + 2, sidx0, si0)
        w_idx(didx0, di0)
        scatter(didx0, pbuf0, s0)
        wait_p(pbuf1, s1)
        fi_dst(k1, didx1, di1)
        w_idx(sidx1, si1)
        gather(sidx1, pbuf1, g1)
        wait_p(pbuf1, g1)
        fi_src(k1 + 2, sidx1, si1)
        w_idx(didx1, di1)
        scatter(didx1, pbuf1, s1)
        wait_p(pbuf0, s0)
        fi_dst(k0 + 2, didx0, di0)
        w_idx(sidx0, si0)
        gather(sidx0, pbuf0, g0)
        return carry

    lax.fori_loop(0, kbl, body, 0)
    wait_p(pbuf0, g0)
    w_idx(didx0, di0)
    scatter(didx0, pbuf0, s0)
    wait_p(pbuf1, s1)
    fi_dst(cl - 1, didx1, di1)
    w_idx(sidx1, si1)
    gather(sidx1, pbuf1, g1)
    wait_p(pbuf1, g1)
    w_idx(didx1, di1)
    scatter(didx1, pbuf1, s1)
    wait_p(pbuf0, s0)
    wait_p(pbuf1, s1)
    plsc.subcore_barrier()
    obase = cid * NP + row0
    for r in range(RPT // CH):
        pltpu.sync_copy(acc_sp.at[pl.ds(row0 + r * CH, CH)], pbuf0)
        pltpu.sync_copy(pbuf0, s2_hbm.at[pl.ds(obase + r * CH, CH)])


@functools.partial(
    pl.kernel,
    out_type=jax.ShapeDtypeStruct((EP, EDIM), _F32),
    mesh=_MESH,
    scratch_types=[
        pltpu.VMEM((EPT,), jnp.int32),
        pltpu.VMEM((EPT,), jnp.int32),
        pltpu.VMEM((CH3, DIN), _F32),
        pltpu.VMEM((CH3, DIN), _F32),
        pltpu.VMEM((CH3, DIN), _F32),
        pltpu.VMEM((CH3, DIN), _F32),
        pltpu.VMEM((CH3, EDIM), _F32),
        pltpu.VMEM((CH3, EDIM), _F32),
        pltpu.SemaphoreType.DMA,
        pltpu.SemaphoreType.DMA,
        pltpu.SemaphoreType.DMA,
        pltpu.SemaphoreType.DMA,
        pltpu.SemaphoreType.DMA,
        pltpu.SemaphoreType.DMA,
    ],
)
def _sc_score(ab_hbm, src_hbm, dst_hbm, out_hbm,
              sidx, didx, u0, u1, v0, v1, o0, o1,
              gu0, gu1, gv0, gv1, so0, so1):
    cid = lax.axis_index("c")
    sid = lax.axis_index("s")
    wid = cid * NS + sid
    ebase = wid * EPT
    pltpu.sync_copy(src_hbm.at[pl.ds(ebase, EPT)], sidx)
    pltpu.sync_copy(dst_hbm.at[pl.ds(ebase, EPT)], didx)

    def gu(k, buf, sem):
        pltpu.async_copy(ab_hbm.at[sidx.at[pl.ds(k * CH3, CH3)]], buf, sem)

    def gv(k, buf, sem):
        pltpu.async_copy(ab_hbm.at[didx.at[pl.ds(k * CH3, CH3)]], buf, sem)

    def ostore(k, buf, sem):
        pltpu.async_copy(buf, out_hbm.at[pl.ds(ebase + k * CH3, CH3)], sem)

    def wait_w(buf, sem):
        pltpu.make_async_copy(ab_hbm.at[pl.ds(0, CH3)], buf, sem).wait()

    def wait_o(buf, sem):
        pltpu.make_async_copy(out_hbm.at[pl.ds(0, CH3)], buf, sem).wait()

    def combine(ob, ub, vb):
        for r in range(CH3):
            ob[r, :] = ub[r, pl.ds(0, EDIM)] + vb[r, pl.ds(EDIM, EDIM)]

    gu(0, u0, gu0)
    gv(0, v0, gv0)
    gu(1, u1, gu1)
    gv(1, v1, gv1)
    wait_w(u0, gu0)
    wait_w(v0, gv0)
    combine(o0, u0, v0)
    ostore(0, o0, so0)
    gu(2, u0, gu0)
    gv(2, v0, gv0)
    wait_w(u1, gu1)
    wait_w(v1, gv1)
    combine(o1, u1, v1)
    ostore(1, o1, so1)
    gu(3, u1, gu1)
    gv(3, v1, gv1)

    def body(kk, carry):
        k0 = 2 * kk
        k1 = k0 + 1
        wait_w(u0, gu0)
        wait_w(v0, gv0)
        wait_o(o0, so0)
        combine(o0, u0, v0)
        ostore(k0, o0, so0)
        gu(k0 + 2, u0, gu0)
        gv(k0 + 2, v0, gv0)
        wait_w(u1, gu1)
        wait_w(v1, gv1)
        wait_o(o1, so1)
        combine(o1, u1, v1)
        ostore(k1, o1, so1)
        gu(k1 + 2, u1, gu1)
        gv(k1 + 2, v1, gv1)
        return carry

    lax.fori_loop(1, CHUNKS3 // 2 - 1, body, 0)
    wait_w(u0, gu0)
    wait_w(v0, gv0)
    wait_o(o0, so0)
    combine(o0, u0, v0)
    ostore(CHUNKS3 - 2, o0, so0)
    wait_w(u1, gu1)
    wait_w(v1, gv1)
    wait_o(o1, so1)
    combine(o1, u1, v1)
    ostore(CHUNKS3 - 1, o1, so1)
    wait_o(o0, so0)
    wait_o(o1, so1)


def kernel(nfeats, efeats, edge_index, Wm1, bm1, Wa1, ba1,
           Wm2, bm2, Wa2, ba2, Wp, bp):
    h0 = nfeats.reshape(N, DIN)
    h0p = jnp.pad(h0, ((0, NP - N), (0, 0)))
    dummy = N + (jnp.arange(EP - E, dtype=jnp.int32) % (NP - N))
    srcp = jnp.concatenate([edge_index[0], dummy])
    dstp = jnp.concatenate([edge_index[1], dummy])
    pad3 = (CA3 - CB3) * CH3
    src2d = jnp.concatenate(
        [srcp, jnp.full((pad3,), N, jnp.int32)]).reshape(-1, CH3)
    dst2d = jnp.concatenate(
        [dstp, jnp.full((pad3,), N, jnp.int32)]).reshape(-1, CH3)

    wm1t, wm1b = Wm1[:DIN], Wm1[DIN:]
    wa1t, wa1b = Wa1[:DIN], Wa1[DIN:]
    wp2 = Wm2[:EDIM] + 0.5 * Wm2[EDIM:]
    wq2 = 0.5 * Wm2[EDIM:]
    wa2t, wa2b = Wa2[:EDIM], Wa2[EDIM:]
    wpt = jnp.pad(Wp[:DOUT], ((0, 0), (0, 16 - NCLS)))
    wpb = jnp.pad(Wp[DOUT:], ((0, 0), (0, 16 - NCLS)))
    bpp = jnp.pad(bp, (0, 16 - NCLS)).reshape(1, 16)
    bm1r = bm1.reshape(1, EDIM)
    ba1r = ba1.reshape(1, EDIM)
    bm2r = bm2.reshape(1, DOUT)
    ba2r = ba2.reshape(1, DOUT)

    e8 = jnp.pad(efeats.reshape(E // 8, 8 * EDIM),
                 ((0, EP8 - E // 8), (0, 0)))
    wblk = jnp.kron(jnp.eye(8, dtype=_F32), wm1b)
    btile = jnp.tile(bm1, 8).reshape(1, 8 * EDIM)
    be8 = 2560
    ew = pl.pallas_call(
        _edge_msg_body,
        grid=(EP8 // be8,),
        in_specs=[pl.BlockSpec((be8, 8 * EDIM), lambda i: (i, 0)),
                  pl.BlockSpec((8 * EDIM, 8 * EDIM), lambda i: (0, 0)),
                  pl.BlockSpec((1, 8 * EDIM), lambda i: (0, 0))],
        out_specs=pl.BlockSpec((be8, 8 * EDIM), lambda i: (i, 0)),
        out_shape=jax.ShapeDtypeStruct((EP8, 8 * EDIM), _F32),
    )(e8, wblk, btile)

    nb = 1280
    hc = pl.pallas_call(
        _node_msg_body,
        grid=(NP // nb,),
        in_specs=[pl.BlockSpec((nb, DIN), lambda i: (i, 0)),
                  pl.BlockSpec((DIN, EDIM), lambda i: (0, 0))],
        out_specs=pl.BlockSpec((nb, DIN), lambda i: (i, 0)),
        out_shape=jax.ShapeDtypeStruct((NP, DIN), _F32),
    )(h0p, wm1t)

    s1 = _sc_layer1(hc, ew, srcp, dstp).reshape(NC, NP, DIN)

    h1, pmat, qb = pl.pallas_call(
        _layer1_apply_body,
        grid=(NP // nb,),
        in_specs=[pl.BlockSpec((nb, DIN), lambda i: (i, 0)),
                  pl.BlockSpec((NC, nb, DIN), lambda i: (0, i, 0)),
                  pl.BlockSpec((DIN, EDIM), lambda i: (0, 0)),
                  pl.BlockSpec((EDIM, EDIM), lambda i: (0, 0)),
                  pl.BlockSpec((1, EDIM), lambda i: (0, 0)),
                  pl.BlockSpec((EDIM, DOUT), lambda i: (0, 0)),
                  pl.BlockSpec((EDIM, DOUT), lambda i: (0, 0)),
                  pl.BlockSpec((1, DOUT), lambda i: (0, 0))],
        out_specs=[pl.BlockSpec((nb, EDIM), lambda i: (i, 0)),
                   pl.BlockSpec((nb, DOUT), lambda i: (i, 0)),
                   pl.BlockSpec((nb, DOUT), lambda i: (i, 0))],
        out_shape=[jax.ShapeDtypeStruct((NP, EDIM), _F32),
                   jax.ShapeDtypeStruct((NP, DOUT), _F32),
                   jax.ShapeDtypeStruct((NP, DOUT), _F32)],
    )(h0p, s1, wa1t, wa1b, ba1r, wp2, wq2, bm2r)

    s2 = _sc_layer2(pmat, srcp, dstp).reshape(NC, NP, DOUT)

    ab = pl.pallas_call(
        _layer2_apply_body,
        grid=(NP // nb,),
        in_specs=[pl.BlockSpec((nb, EDIM), lambda i: (i, 0)),
                  pl.BlockSpec((NC, nb, DIN), lambda i: (0, i, 0)),
                  pl.BlockSpec((NC, nb, DOUT), lambda i: (0, i, 0)),
                  pl.BlockSpec((nb, DOUT), lambda i: (i, 0)),
                  pl.BlockSpec((EDIM, DOUT), lambda i: (0, 0)),
                  pl.BlockSpec((DOUT, DOUT), lambda i: (0, 0)),
                  pl.BlockSpec((1, DOUT), lambda i: (0, 0)),
                  pl.BlockSpec((DOUT, 16), lambda i: (0, 0)),
                  pl.BlockSpec((DOUT, 16), lambda i: (0, 0)),
                  pl.BlockSpec((1, 16), lambda i: (0, 0))],
        out_specs=pl.BlockSpec((nb, DIN), lambda i: (i, 0)),
        out_shape=jax.ShapeDtypeStruct((NP, DIN), _F32),
    )(h1, s1, s2, qb, wa2t, wa2b, ba2r, wpt, wpb, bpp)

    score = _sc_score(ab, srcp, dstp)
    return score[:E, :NCLS]


# speedup vs baseline: 3.5562x; 1.0024x over previous
import functools
import jax
import jax.numpy as jnp
from jax import lax
from jax.experimental import pallas as pl
from jax.experimental.pallas import tpu as pltpu
from jax.experimental.pallas import tpu_sc as plsc

N = 10000
E = 320000
DIN = 128
EDIM = 16
DOUT = 128
NCLS = 11

NC = 2
NS = 16
NW = NC * NS
CH = 128
CHUNKS = 80
CH3 = 64
CHUNKS3 = 160
EPT = CH * CHUNKS
EP = EPT * NW
NP = 10240
RPT = NP // NS
KB = CHUNKS // 2 - 1
KB3 = CHUNKS3 // 2 - 1
CA = 80
CB = 80
CA3 = 80
CB3 = 80
EP8 = EP // 8

_MESH = plsc.VectorSubcoreMesh(core_axis_name="c", subcore_axis_name="s")
_F32 = jnp.float32
_HIGH = lax.Precision.HIGHEST


def _dot(a, b):
    return jnp.dot(a, b, preferred_element_type=_F32, precision=_HIGH)


def _edge_msg_body(e_ref, w_ref, b_ref, o_ref):
    o_ref[...] = _dot(e_ref[...], w_ref[...]) + b_ref[...]


def _node_msg_body(h_ref, w_ref, o_ref):
    nb = h_ref.shape[0]
    hw = _dot(h_ref[...], w_ref[...])
    col = lax.broadcasted_iota(jnp.int32, (nb, DIN), 1)
    wide = jnp.concatenate(
        [hw, jnp.zeros((nb, DIN - EDIM), _F32)], axis=1)
    o_ref[...] = jnp.where(col == EDIM, 1.0, wide)


def _layer1_apply_body(h0_ref, s1_ref, wa1t_ref, wa1b_ref, ba1_ref,
                       wp2_ref, wq2_ref, bm2_ref, h1_ref, p_ref, qb_ref):
    acc = s1_ref[0] + s1_ref[1]
    cnt = acc[:, EDIM]
    inv = 1.0 / jnp.maximum(cnt, 1.0)
    hn1 = acc[:, :EDIM] * inv[:, None]
    h1 = jax.nn.relu(_dot(h0_ref[...], wa1t_ref[...]) +
                     _dot(hn1, wa1b_ref[...]) + ba1_ref[...])
    h1_ref[...] = h1
    p_ref[...] = _dot(h1, wp2_ref[...])
    qb_ref[...] = _dot(h1, wq2_ref[...]) + bm2_ref[...]


def _layer2_apply_body(h1_ref, s1_ref, s2_ref, qb_ref, wa2t_ref, wa2b_ref,
                       ba2_ref, wpt_ref, wpb_ref, bp_ref, ab_ref):
    nb = h1_ref.shape[0]
    cnt = s1_ref[0][:, EDIM] + s1_ref[1][:, EDIM]
    inv = 1.0 / jnp.maximum(cnt, 1.0)
    gate = jnp.minimum(cnt, 1.0)
    s2t = s2_ref[0] + s2_ref[1]
    hn2 = s2t * inv[:, None] + qb_ref[...] * gate[:, None]
    h2 = jax.nn.relu(_dot(h1_ref[...], wa2t_ref[...]) +
                     _dot(hn2, wa2b_ref[...]) + ba2_ref[...])
    a = _dot(h2, wpt_ref[...]) + bp_ref[...]
    b = _dot(h2, wpb_ref[...])
    ab_ref[...] = jnp.concatenate(
        [a, b, jnp.zeros((nb, DIN - 2 * EDIM), _F32)], axis=1)


def _zero_rows(buf, rows, width):
    zero = jnp.zeros((16,), _F32)
    for r in range(rows):
        for c in range(width // 16):
            buf[r, pl.ds(c * 16, 16)] = zero


@functools.partial(
    pl.kernel,
    out_type=jax.ShapeDtypeStruct((NC * NP, DIN), _F32),
    mesh=_MESH,
    scratch_types=[
        pltpu.VMEM((CH,), jnp.int32),
        pltpu.VMEM((CH,), jnp.int32),
        pltpu.VMEM((CH,), jnp.int32),
        pltpu.VMEM((CH,), jnp.int32),
        pltpu.VMEM((CH, DIN), _F32),
        pltpu.VMEM((CH, DIN), _F32),
        pltpu.VMEM((CH // 8, 8 * EDIM), _F32),
        pltpu.VMEM((CH // 8, 8 * EDIM), _F32),
        pltpu.VMEM_SHARED((NP, DIN), _F32),
        pltpu.SemaphoreType.DMA,
        pltpu.SemaphoreType.DMA,
        pltpu.SemaphoreType.DMA,
        pltpu.SemaphoreType.DMA,
        pltpu.SemaphoreType.DMA,
        pltpu.SemaphoreType.DMA,
        pltpu.SemaphoreType.DMA,
        pltpu.SemaphoreType.DMA,
        pltpu.SemaphoreType.DMA,
        pltpu.SemaphoreType.DMA,
    ],
)
def _sc_layer1(hc_hbm, ew_hbm, src_hbm, dst_hbm, s1_hbm,
               sidx0, sidx1, didx0, didx1, pbuf0, pbuf1, ebuf0, ebuf1,
               acc_sp, g0, g1, s0, s1, e0, e1, si0, si1, di0, di1):
    cid = lax.axis_index("c")
    sid = lax.axis_index("s")
    wid = cid * NS + sid
    _zero_rows(pbuf0, CH, DIN)
    _zero_rows(pbuf1, CH, DIN)
    row0 = sid * RPT
    for r in range(RPT // CH):
        pltpu.sync_copy(pbuf0, acc_sp.at[pl.ds(row0 + r * CH, CH)])
    koff = jnp.where(cid == 0, sid * CA, NS * CA + sid * CB)
    cl = jnp.where(cid == 0, CA, CB)
    kbl = cl // 2 - 1
    ebase = koff * CH
    ebase8 = koff * (CH // 8)
    pltpu.sync_copy(src_hbm.at[pl.ds(ebase, CH)], sidx0)
    plsc.subcore_barrier()

    def fi_src(k, buf, sem):
        pltpu.async_copy(src_hbm.at[pl.ds(ebase + k * CH, CH)], buf, sem)

    def fi_dst(k, buf, sem):
        pltpu.async_copy(dst_hbm.at[pl.ds(ebase + k * CH, CH)], buf, sem)

    def w_idx(buf, sem):
        pltpu.make_async_copy(src_hbm.at[pl.ds(0, CH)], buf, sem).wait()

    def gather(sbuf, buf, sem):
        pltpu.async_copy(hc_hbm.at[sbuf], buf, sem)

    def scatter(dbuf, buf, sem):
        pltpu.async_copy(buf, acc_sp.at[dbuf], sem, add=True)

    def eload(k, buf, sem):
        pltpu.async_copy(ew_hbm.at[pl.ds(ebase8 + k * (CH // 8), CH // 8)],
                         buf, sem)

    def wait_p(buf, sem):
        pltpu.make_async_copy(hc_hbm.at[pl.ds(0, CH)], buf, sem).wait()

    def wait_e(buf, sem):
        pltpu.make_async_copy(ew_hbm.at[pl.ds(0, CH // 8)], buf, sem).wait()

    def addin(pbuf, ebuf):
        for r in range(CH):
            pbuf[r, pl.ds(0, EDIM)] = (
                pbuf[r, pl.ds(0, EDIM)]
                + ebuf[r // 8, pl.ds((r % 8) * EDIM, EDIM)])

    gather(sidx0, pbuf0, g0)
    fi_dst(0, didx0, di0)
    fi_src(1, sidx1, si1)
    eload(0, ebuf0, e0)
    eload(1, ebuf1, e1)
    pltpu.async_copy(pbuf1, acc_sp.at[sidx0], s1, add=True)

    def body(kk, carry):
        k0 = 2 * kk
        k1 = k0 + 1
        wait_p(pbuf0, g0)
        fi_src(k0 + 2, sidx0, si0)
        wait_e(ebuf0, e0)
        addin(pbuf0, ebuf0)
        eload(k0 + 2, ebuf0, e0)
        w_idx(didx0, di0)
        scatter(didx0, pbuf0, s0)
        wait_p(pbuf1, s1)
        fi_dst(k1, didx1, di1)
        w_idx(sidx1, si1)
        gather(sidx1, pbuf1, g1)
        wait_p(pbuf1, g1)
        fi_src(k1 + 2, sidx1, si1)
        wait_e(ebuf1, e1)
        addin(pbuf1, ebuf1)
        eload(k1 + 2, ebuf1, e1)
        w_idx(didx1, di1)
        scatter(didx1, pbuf1, s1)
        wait_p(pbuf0, s0)
        fi_dst(k0 + 2, didx0, di0)
        w_idx(sidx0, si0)
        gather(sidx0, pbuf0, g0)
        return carry

    lax.fori_loop(0, kbl, body, 0)
    wait_p(pbuf0, g0)
    wait_e(ebuf0, e0)
    addin(pbuf0, ebuf0)
    w_idx(didx0, di0)
    scatter(didx0, pbuf0, s0)
    wait_p(pbuf1, s1)
    fi_dst(cl - 1, didx1, di1)
    w_idx(sidx1, si1)
    gather(sidx1, pbuf1, g1)
    wait_p(pbuf1, g1)
    wait_e(ebuf1, e1)
    addin(pbuf1, ebuf1)
    w_idx(didx1, di1)
    scatter(didx1, pbuf1, s1)
    wait_p(pbuf0, s0)
    wait_p(pbuf1, s1)
    plsc.subcore_barrier()
    obase = cid * NP + row0
    for r in range(RPT // CH):
        pltpu.sync_copy(acc_sp.at[pl.ds(row0 + r * CH, CH)], pbuf0)
        pltpu.sync_copy(pbuf0, s1_hbm.at[pl.ds(obase + r * CH, CH)])


@functools.partial(
    pl.kernel,
    out_type=jax.ShapeDtypeStruct((NC * NP, DOUT), _F32),
    mesh=_MESH,
    scratch_types=[
        pltpu.VMEM((CH,), jnp.int32),
        pltpu.VMEM((CH,), jnp.int32),
        pltpu.VMEM((CH,), jnp.int32),
        pltpu.VMEM((CH,), jnp.int32),
        pltpu.VMEM((CH, DOUT), _F32),
        pltpu.VMEM((CH, DOUT), _F32),
        pltpu.VMEM_SHARED((NP, DOUT), _F32),
        pltpu.SemaphoreType.DMA,
        pltpu.SemaphoreType.DMA,
        pltpu.SemaphoreType.DMA,
        pltpu.SemaphoreType.DMA,
        pltpu.SemaphoreType.DMA,
        pltpu.SemaphoreType.DMA,
        pltpu.SemaphoreType.DMA,
        pltpu.SemaphoreType.DMA,
    ],
)
def _sc_layer2(p_hbm, src_hbm, dst_hbm, s2_hbm,
               sidx0, sidx1, didx0, didx1, pbuf0, pbuf1,
               acc_sp, g0, g1, s0, s1, si0, si1, di0, di1):
    cid = lax.axis_index("c")
    sid = lax.axis_index("s")
    wid = cid * NS + sid
    _zero_rows(pbuf0, CH, DOUT)
    _zero_rows(pbuf1, CH, DOUT)
    row0 = sid * RPT
    for r in range(RPT // CH):
        pltpu.sync_copy(pbuf0, acc_sp.at[pl.ds(row0 + r * CH, CH)])
    koff = jnp.where(cid == 0, sid * CA, NS * CA + sid * CB)
    cl = jnp.where(cid == 0, CA, CB)
    kbl = cl // 2 - 1
    ebase = koff * CH
    pltpu.sync_copy(src_hbm.at[pl.ds(ebase, CH)], sidx0)
    plsc.subcore_barrier()

    def fi_src(k, buf, sem):
        pltpu.async_copy(src_hbm.at[pl.ds(ebase + k * CH, CH)], buf, sem)

    def fi_dst(k, buf, sem):
        pltpu.async_copy(dst_hbm.at[pl.ds(ebase + k * CH, CH)], buf, sem)

    def w_idx(buf, sem):
        pltpu.make_async_copy(src_hbm.at[pl.ds(0, CH)], buf, sem).wait()

    def gather(sbuf, buf, sem):
        pltpu.async_copy(p_hbm.at[sbuf], buf, sem)

    def scatter(dbuf, buf, sem):
        pltpu.async_copy(buf, acc_sp.at[dbuf], sem, add=True)

    def wait_p(buf, sem):
        pltpu.make_async_copy(p_hbm.at[pl.ds(0, CH)], buf, sem).wait()

    gather(sidx0, pbuf0, g0)
    fi_dst(0, didx0, di0)
    fi_src(1, sidx1, si1)
    pltpu.async_copy(pbuf1, acc_sp.at[sidx0], s1, add=True)

    def body(kk, carry):
        k0 = 2 * kk
        k1 = k0 + 1
        wait_p(pbuf0, g0)
        fi_src(k0 + 2, sidx0, si0)
        w_idx(didx0, di0)
        scatter(didx0, pbuf0, s0)
        wait_p(pbuf1, s1)
        fi_dst(k1, didx1, di1)
        w_idx(sidx1, si1)
        gather(sidx1, pbuf1, g1)
        wait_p(pbuf1, g1)
        fi_src(k1 + 2, sidx1, si1)
        w_idx(didx1, di1)
        scatter(didx1, pbuf1, s1)
        wait_p(pbuf0, s0)
        fi_dst(k0 + 2, didx0, di0)
        w_idx(sidx0, si0)
        gather(sidx0, pbuf0, g0)
        return carry

    lax.fori_loop(0, kbl, body, 0)
    wait_p(pbuf0, g0)
    w_idx(didx0, di0)
    scatter(didx0, pbuf0, s0)
    wait_p(pbuf1, s1)
    fi_dst(cl - 1, didx1, di1)
    w_idx(sidx1, si1)
    gather(sidx1, pbuf1, g1)
    wait_p(pbuf1, g1)
    w_idx(didx1, di1)
    scatter(didx1, pbuf1, s1)
    wait_p(pbuf0, s0)
    wait_p(pbuf1, s1)
    plsc.subcore_barrier()
    obase = cid * NP + row0
    for r in range(RPT // CH):
        pltpu.sync_copy(acc_sp.at[pl.ds(row0 + r * CH, CH)], pbuf0)
        pltpu.sync_copy(pbuf0, s2_hbm.at[pl.ds(obase + r * CH, CH)])


@functools.partial(
    pl.kernel,
    out_type=jax.ShapeDtypeStruct((EP, EDIM), _F32),
    mesh=_MESH,
    scratch_types=[
        pltpu.VMEM((EPT,), jnp.int32),
        pltpu.VMEM((EPT,), jnp.int32),
        pltpu.VMEM((CH3, DIN), _F32),
        pltpu.VMEM((CH3, DIN), _F32),
        pltpu.VMEM((CH3, DIN), _F32),
        pltpu.VMEM((CH3, DIN), _F32),
        pltpu.VMEM((CH3, EDIM), _F32),
        pltpu.VMEM((CH3, EDIM), _F32),
        pltpu.SemaphoreType.DMA,
        pltpu.SemaphoreType.DMA,
        pltpu.SemaphoreType.DMA,
        pltpu.SemaphoreType.DMA,
        pltpu.SemaphoreType.DMA,
        pltpu.SemaphoreType.DMA,
    ],
)
def _sc_score(ab_hbm, src_hbm, dst_hbm, out_hbm,
              sidx, didx, u0, u1, v0, v1, o0, o1,
              gu0, gu1, gv0, gv1, so0, so1):
    cid = lax.axis_index("c")
    sid = lax.axis_index("s")
    wid = cid * NS + sid
    ebase = wid * EPT
    pltpu.sync_copy(src_hbm.at[pl.ds(ebase, EPT)], sidx)
    pltpu.sync_copy(dst_hbm.at[pl.ds(ebase, EPT)], didx)

    def gu(k, buf, sem):
        pltpu.async_copy(ab_hbm.at[sidx.at[pl.ds(k * CH3, CH3)]], buf, sem)

    def gv(k, buf, sem):
        pltpu.async_copy(ab_hbm.at[didx.at[pl.ds(k * CH3, CH3)]], buf, sem)

    def ostore(k, buf, sem):
        pltpu.async_copy(buf, out_hbm.at[pl.ds(ebase + k * CH3, CH3)], sem)

    def wait_w(buf, sem):
        pltpu.make_async_copy(ab_hbm.at[pl.ds(0, CH3)], buf, sem).wait()

    def wait_o(buf, sem):
        pltpu.make_async_copy(out_hbm.at[pl.ds(0, CH3)], buf, sem).wait()

    def combine(ob, ub, vb):
        for r in range(CH3):
            ob[r, :] = ub[r, pl.ds(0, EDIM)] + vb[r, pl.ds(EDIM, EDIM)]

    gu(0, u0, gu0)
    gv(0, v0, gv0)
    gu(1, u1, gu1)
    gv(1, v1, gv1)
    wait_w(u0, gu0)
    wait_w(v0, gv0)
    combine(o0, u0, v0)
    ostore(0, o0, so0)
    gu(2, u0, gu0)
    gv(2, v0, gv0)
    wait_w(u1, gu1)
    wait_w(v1, gv1)
    combine(o1, u1, v1)
    ostore(1, o1, so1)
    gu(3, u1, gu1)
    gv(3, v1, gv1)

    def body(kk, carry):
        k0 = 2 * kk
        k1 = k0 + 1
        wait_w(u0, gu0)
        wait_w(v0, gv0)
        wait_o(o0, so0)
        combine(o0, u0, v0)
        ostore(k0, o0, so0)
        gu(k0 + 2, u0, gu0)
        gv(k0 + 2, v0, gv0)
        wait_w(u1, gu1)
        wait_w(v1, gv1)
        wait_o(o1, so1)
        combine(o1, u1, v1)
        ostore(k1, o1, so1)
        gu(k1 + 2, u1, gu1)
        gv(k1 + 2, v1, gv1)
        return carry

    lax.fori_loop(1, CHUNKS3 // 2 - 1, body, 0)
    wait_w(u0, gu0)
    wait_w(v0, gv0)
    wait_o(o0, so0)
    combine(o0, u0, v0)
    ostore(CHUNKS3 - 2, o0, so0)
    wait_w(u1, gu1)
    wait_w(v1, gv1)
    wait_o(o1, so1)
    combine(o1, u1, v1)
    ostore(CHUNKS3 - 1, o1, so1)
    wait_o(o0, so0)
    wait_o(o1, so1)


def kernel(nfeats, efeats, edge_index, Wm1, bm1, Wa1, ba1,
           Wm2, bm2, Wa2, ba2, Wp, bp):
    h0 = nfeats.reshape(N, DIN)
    h0p = jnp.pad(h0, ((0, NP - N), (0, 0)))
    dummy = N + (jnp.arange(EP - E, dtype=jnp.int32) % (NP - N))
    srcp = jnp.concatenate([edge_index[0], dummy])
    dstp = jnp.concatenate([edge_index[1], dummy])
    pad3 = (CA3 - CB3) * CH3
    src2d = jnp.concatenate(
        [srcp, jnp.full((pad3,), N, jnp.int32)]).reshape(-1, CH3)
    dst2d = jnp.concatenate(
        [dstp, jnp.full((pad3,), N, jnp.int32)]).reshape(-1, CH3)

    wm1t, wm1b = Wm1[:DIN], Wm1[DIN:]
    wa1t, wa1b = Wa1[:DIN], Wa1[DIN:]
    wp2 = Wm2[:EDIM] + 0.5 * Wm2[EDIM:]
    wq2 = 0.5 * Wm2[EDIM:]
    wa2t, wa2b = Wa2[:EDIM], Wa2[EDIM:]
    wpt = jnp.pad(Wp[:DOUT], ((0, 0), (0, 16 - NCLS)))
    wpb = jnp.pad(Wp[DOUT:], ((0, 0), (0, 16 - NCLS)))
    bpp = jnp.pad(bp, (0, 16 - NCLS)).reshape(1, 16)
    bm1r = bm1.reshape(1, EDIM)
    ba1r = ba1.reshape(1, EDIM)
    bm2r = bm2.reshape(1, DOUT)
    ba2r = ba2.reshape(1, DOUT)

    e8 = jnp.pad(efeats.reshape(E // 8, 8 * EDIM),
                 ((0, EP8 - E // 8), (0, 0)))
    wblk = jnp.kron(jnp.eye(8, dtype=_F32), wm1b)
    btile = jnp.tile(bm1, 8).reshape(1, 8 * EDIM)
    be8 = 2560
    ew = pl.pallas_call(
        _edge_msg_body,
        grid=(EP8 // be8,),
        in_specs=[pl.BlockSpec((be8, 8 * EDIM), lambda i: (i, 0)),
                  pl.BlockSpec((8 * EDIM, 8 * EDIM), lambda i: (0, 0)),
                  pl.BlockSpec((1, 8 * EDIM), lambda i: (0, 0))],
        out_specs=pl.BlockSpec((be8, 8 * EDIM), lambda i: (i, 0)),
        out_shape=jax.ShapeDtypeStruct((EP8, 8 * EDIM), _F32),
    )(e8, wblk, btile)

    nb = 1280
    hc = pl.pallas_call(
        _node_msg_body,
        grid=(NP // nb,),
        in_specs=[pl.BlockSpec((nb, DIN), lambda i: (i, 0)),
                  pl.BlockSpec((DIN, EDIM), lambda i: (0, 0))],
        out_specs=pl.BlockSpec((nb, DIN), lambda i: (i, 0)),
        out_shape=jax.ShapeDtypeStruct((NP, DIN), _F32),
    )(h0p, wm1t)

    s1 = _sc_layer1(hc, ew, srcp, dstp).reshape(NC, NP, DIN)

    h1, pmat, qb = pl.pallas_call(
        _layer1_apply_body,
        grid=(NP // nb,),
        in_specs=[pl.BlockSpec((nb, DIN), lambda i: (i, 0)),
                  pl.BlockSpec((NC, nb, DIN), lambda i: (0, i, 0)),
                  pl.BlockSpec((DIN, EDIM), lambda i: (0, 0)),
                  pl.BlockSpec((EDIM, EDIM), lambda i: (0, 0)),
                  pl.BlockSpec((1, EDIM), lambda i: (0, 0)),
                  pl.BlockSpec((EDIM, DOUT), lambda i: (0, 0)),
                  pl.BlockSpec((EDIM, DOUT), lambda i: (0, 0)),
                  pl.BlockSpec((1, DOUT), lambda i: (0, 0))],
        out_specs=[pl.BlockSpec((nb, EDIM), lambda i: (i, 0)),
                   pl.BlockSpec((nb, DOUT), lambda i: (i, 0)),
                   pl.BlockSpec((nb, DOUT), lambda i: (i, 0))],
        out_shape=[jax.ShapeDtypeStruct((NP, EDIM), _F32),
                   jax.ShapeDtypeStruct((NP, DOUT), _F32),
                   jax.ShapeDtypeStruct((NP, DOUT), _F32)],
    )(h0p, s1, wa1t, wa1b, ba1r, wp2, wq2, bm2r)

    s2 = _sc_layer2(pmat, srcp, dstp).reshape(NC, NP, DOUT)

    ab = pl.pallas_call(
        _layer2_apply_body,
        grid=(NP // nb,),
        in_specs=[pl.BlockSpec((nb, EDIM), lambda i: (i, 0)),
                  pl.BlockSpec((NC, nb, DIN), lambda i: (0, i, 0)),
                  pl.BlockSpec((NC, nb, DOUT), lambda i: (0, i, 0)),
                  pl.BlockSpec((nb, DOUT), lambda i: (i, 0)),
                  pl.BlockSpec((EDIM, DOUT), lambda i: (0, 0)),
                  pl.BlockSpec((DOUT, DOUT), lambda i: (0, 0)),
                  pl.BlockSpec((1, DOUT), lambda i: (0, 0)),
                  pl.BlockSpec((DOUT, 16), lambda i: (0, 0)),
                  pl.BlockSpec((DOUT, 16), lambda i: (0, 0)),
                  pl.BlockSpec((1, 16), lambda i: (0, 0))],
        out_specs=pl.BlockSpec((nb, DIN), lambda i: (i, 0)),
        out_shape=jax.ShapeDtypeStruct((NP, DIN), _F32),
    )(h1, s1, s2, qb, wa2t, wa2b, ba2r, wpt, wpb, bpp)

    score = _sc_score(ab, srcp, dstp)
    return score[:E, :NCLS]
